# Initial kernel scaffold; baseline (speedup 1.0000x reference)
import jax
import jax.numpy as jnp
from jax.experimental import pallas as pl


def kernel(x, edge_index, batch, coord, params):
    raise NotImplementedError("write your pallas kernel here")


# speedup vs baseline: 22.7539x; 22.7539x over previous
import functools

import jax
import jax.numpy as jnp
from jax import lax
from jax.experimental import pallas as pl
from jax.experimental.pallas import tpu as pltpu
from jax.experimental.pallas import tpu_sc as plsc

N = 10000
E = 320000
D = 128
NP = 12288
EP = 327680
NW = 32
ET = EP // NW
ECH = ET // 128
ERW = EP // 128
NSL = NP // 16
NT = NP // NW
K0 = 100
BR = 1024
GRID = NP // BR
BNI = float(1.0 / (1.0 + 1e-5) ** 0.5)

f32 = jnp.float32
i32 = jnp.int32

@functools.cache
def _mesh():
    return plsc.VectorSubcoreMesh(core_axis_name="c", subcore_axis_name="s",
                                  num_cores=2, num_subcores=16)


def _fiota(shape, dim):
    return lax.broadcasted_iota(i32, shape, dim).astype(f32)


def _wid():
    return lax.axis_index("s") * 2 + lax.axis_index("c")


def _zero16(ref, n):
    def b(t, _):
        ref[pl.ds(t * 16, 16)] = jnp.zeros((16,), f32)
        return 0
    lax.fori_loop(0, n // 16, b, 0)


def _zero2d(ref, rows, cols):
    def b(t, _):
        r = t // (cols // 16)
        c = t % (cols // 16)
        ref[r, pl.ds(c * 16, 16)] = jnp.zeros((16,), f32)
        return 0
    lax.fori_loop(0, rows * (cols // 16), b, 0)


def _pa_body(s2_h, d2_h, ds2_h, asrc_h, adst_h,
             ex_o, s_o, hist_o,
             s2v, d2v, ds2v, asv, adv, exv, onesv, zb, s_sh, h_sh):
    cid = lax.axis_index("c")
    sid = lax.axis_index("s")
    wid = _wid()
    cb = wid * ECH
    pltpu.sync_copy(s2_h.at[pl.ds(cb, ECH)], s2v)
    pltpu.sync_copy(d2_h.at[pl.ds(cb, ECH)], d2v)
    pltpu.sync_copy(ds2_h.at[pl.ds(cb, ECH)], ds2v)
    pltpu.sync_copy(asrc_h, asv)
    pltpu.sync_copy(adst_h, adv)
    _zero16(zb, NSL)
    nb = sid * NSL
    pltpu.sync_copy(zb, s_sh.at[pl.ds(nb, NSL)])
    pltpu.sync_copy(zb, h_sh.at[pl.ds(nb, NSL)])
    _zero16(onesv, ET)

    def ones_b(t, _):
        onesv[pl.ds(t * 16, 16)] = jnp.full((16,), 1.0, f32)
        return 0
    lax.fori_loop(0, ET // 16, ones_b, 0)

    def comp(t, _):
        j = t // 8
        c = t % 8
        s16 = s2v[j, pl.ds(c * 16, 16)]
        d16 = d2v[j, pl.ds(c * 16, 16)]
        a = plsc.load_gather(asv, [s16]) + plsc.load_gather(adv, [d16])
        a = jnp.where(a > 0, a, 0.2 * a)
        exv[pl.ds(t * 16, 16)] = jnp.exp(a)
        return 0
    lax.fori_loop(0, ET // 16, comp, 0)
    plsc.subcore_barrier()

    def scat(j, _):
        pltpu.sync_copy(exv.at[pl.ds(j * 128, 128)], s_sh.at[d2v.at[j]], add=True)
        pltpu.sync_copy(onesv.at[pl.ds(j * 128, 128)], h_sh.at[ds2v.at[j]],
                        add=True)
        return 0
    lax.fori_loop(0, ECH, scat, 0)
    pltpu.sync_copy(exv, ex_o.at[pl.ds(wid * ET, ET)])
    plsc.subcore_barrier()
    pltpu.sync_copy(s_sh.at[pl.ds(nb, NSL)], zb)
    pltpu.sync_copy(zb, s_o.at[cid, pl.ds(nb, NSL)])
    pltpu.sync_copy(h_sh.at[pl.ds(nb, NSL)], zb)
    pltpu.sync_copy(zb, hist_o.at[cid, pl.ds(nb, NSL)])


@functools.cache
def _pass_a():
  return pl.kernel(
    _pa_body,
    out_type=(jax.ShapeDtypeStruct((EP,), f32),
              jax.ShapeDtypeStruct((2, NP), f32),
              jax.ShapeDtypeStruct((2, NP), f32)),
    mesh=_mesh(),
    compiler_params=pltpu.CompilerParams(needs_layout_passes=False),
    scratch_types=[
        pltpu.VMEM((ECH, 128), i32), pltpu.VMEM((ECH, 128), i32),
        pltpu.VMEM((ECH, 128), i32),
        pltpu.VMEM((NP,), f32), pltpu.VMEM((NP,), f32),
        pltpu.VMEM((ET,), f32), pltpu.VMEM((ET,), f32),
        pltpu.VMEM((NSL,), f32),
        pltpu.VMEM_SHARED((NP,), f32), pltpu.VMEM_SHARED((NP,), f32),
    ],
)


def _rows_body(s2_h, d2_h, w_h, tab_h, o_o,
               s2v, d2v, wv, rows, zb, a_sh, sem, sem2):
    cid = lax.axis_index("c")
    sid = lax.axis_index("s")
    wid = _wid()
    cb = pl.multiple_of(wid * ECH, 8)
    pltpu.sync_copy(s2_h.at[pl.ds(cb, ECH)], s2v)
    pltpu.sync_copy(d2_h.at[pl.ds(cb, ECH)], d2v)
    pltpu.sync_copy(w_h.at[pl.ds(pl.multiple_of(wid * ET, 128), ET)], wv)
    _zero2d(zb, 128, 64)
    nb = sid * NSL

    def zrow(r, _):
        pltpu.sync_copy(zb, a_sh.at[pl.ds(pl.multiple_of(nb + r * 128, 128),
                                          128)])
        return 0
    lax.fori_loop(0, NSL // 128, zrow, 0)
    plsc.subcore_barrier()

    def chunk(j, _):
        pltpu.async_copy(tab_h.at[s2v.at[j]], rows, sem).wait()

        def rb(r, _2):
            eb = plsc.load_gather(wv, [jnp.zeros((16,), i32) + (j * 128 + r)])
            for g in range(4):
                rows[r, pl.ds(g * 16, 16)] = rows[r, pl.ds(g * 16, 16)] * eb
            return 0
        lax.fori_loop(0, 128, rb, 0)
        pltpu.async_copy(rows, a_sh.at[d2v.at[j]], sem2, add=True).wait()
        return 0
    lax.fori_loop(0, ECH, chunk, 0)
    plsc.subcore_barrier()

    def wb(r, _):
        off = pl.multiple_of(nb + r * 128, 128)
        pltpu.sync_copy(a_sh.at[pl.ds(off, 128)], rows)
        pltpu.sync_copy(rows, o_o.at[cid, pl.ds(off, 128)])
        return 0
    lax.fori_loop(0, NSL // 128, wb, 0)


@functools.cache
def _pass_rows():
  return pl.kernel(
    _rows_body,
    out_type=jax.ShapeDtypeStruct((2, NP, 64), f32),
    mesh=_mesh(),
    compiler_params=pltpu.CompilerParams(needs_layout_passes=False,
                                         use_tc_tiling_on_sc=False),
    scratch_types=[
        pltpu.VMEM((ECH, 128), i32), pltpu.VMEM((ECH, 128), i32),
        pltpu.VMEM((ET,), f32), pltpu.VMEM((128, 64), f32),
        pltpu.VMEM((128, 64), f32),
        pltpu.VMEM_SHARED((NP, 64), f32),
        pltpu.SemaphoreType.DMA, pltpu.SemaphoreType.DMA,
    ],
)


def _pd_body(s2_h, d2_h, dis_h, z0_h, z1_h,
             nrm_o, agg_o,
             s2v, d2v, disv, z0v, z1v, nv, v0, v1, zb, a0_sh, a1_sh):
    cid = lax.axis_index("c")
    sid = lax.axis_index("s")
    wid = _wid()
    cb = wid * ECH
    pltpu.sync_copy(s2_h.at[pl.ds(cb, ECH)], s2v)
    pltpu.sync_copy(d2_h.at[pl.ds(cb, ECH)], d2v)
    pltpu.sync_copy(dis_h, disv)
    pltpu.sync_copy(z0_h, z0v)
    pltpu.sync_copy(z1_h, z1v)
    _zero16(zb, NSL)
    nb = sid * NSL
    pltpu.sync_copy(zb, a0_sh.at[pl.ds(nb, NSL)])
    pltpu.sync_copy(zb, a1_sh.at[pl.ds(nb, NSL)])

    def comp(t, _):
        j = t // 8
        c = t % 8
        s16 = s2v[j, pl.ds(c * 16, 16)]
        d16 = d2v[j, pl.ds(c * 16, 16)]
        nr = plsc.load_gather(disv, [s16]) * plsc.load_gather(disv, [d16])
        nv[pl.ds(t * 16, 16)] = nr
        v0[pl.ds(t * 16, 16)] = nr * plsc.load_gather(z0v, [s16])
        v1[pl.ds(t * 16, 16)] = nr * plsc.load_gather(z1v, [s16])
        return 0
    lax.fori_loop(0, ET // 16, comp, 0)
    plsc.subcore_barrier()

    def scat(j, _):
        pltpu.sync_copy(v0.at[pl.ds(j * 128, 128)], a0_sh.at[d2v.at[j]], add=True)
        pltpu.sync_copy(v1.at[pl.ds(j * 128, 128)], a1_sh.at[d2v.at[j]], add=True)
        return 0
    lax.fori_loop(0, ECH, scat, 0)
    pltpu.sync_copy(nv, nrm_o.at[pl.ds(wid * ET, ET)])
    plsc.subcore_barrier()
    pltpu.sync_copy(a0_sh.at[pl.ds(nb, NSL)], zb)
    pltpu.sync_copy(zb, agg_o.at[cid, 0, pl.ds(nb, NSL)])
    pltpu.sync_copy(a1_sh.at[pl.ds(nb, NSL)], zb)
    pltpu.sync_copy(zb, agg_o.at[cid, 1, pl.ds(nb, NSL)])


@functools.cache
def _pass_d():
  return pl.kernel(
    _pd_body,
    out_type=(jax.ShapeDtypeStruct((EP,), f32),
              jax.ShapeDtypeStruct((2, 2, NP), f32)),
    mesh=_mesh(),
    compiler_params=pltpu.CompilerParams(needs_layout_passes=False),
    scratch_types=[
        pltpu.VMEM((ECH, 128), i32), pltpu.VMEM((ECH, 128), i32),
        pltpu.VMEM((NP,), f32), pltpu.VMEM((NP,), f32), pltpu.VMEM((NP,), f32),
        pltpu.VMEM((ET,), f32), pltpu.VMEM((ET,), f32), pltpu.VMEM((ET,), f32),
        pltpu.VMEM((NSL,), f32),
        pltpu.VMEM_SHARED((NP,), f32), pltpu.VMEM_SHARED((NP,), f32),
    ],
)


def _pf_body(s2_h, d2_h, cid_h, c0_h, c1_h,
             cnt_o, pool_o,
             s2v, d2v, cidv, ci2v, c0v, c1v, onev, keyv, valv, zb,
             cnt_sh, p0_sh, p1_sh, pc_sh):
    cid = lax.axis_index("c")
    sid = lax.axis_index("s")
    wid = _wid()
    cb = wid * ECH
    pltpu.sync_copy(s2_h.at[pl.ds(cb, ECH)], s2v)
    pltpu.sync_copy(d2_h.at[pl.ds(cb, ECH)], d2v)
    pltpu.sync_copy(cid_h, cidv)
    nt0 = pl.multiple_of(wid * NT, 128)
    pltpu.sync_copy(c0_h.at[pl.ds(nt0, NT)], c0v)
    pltpu.sync_copy(c1_h.at[pl.ds(nt0, NT)], c1v)

    def ci_b(t, _):
        v16 = cidv[pl.ds(pl.multiple_of(nt0 + t * 16, 16), 16)]
        ci2v[t // 8, pl.ds((t % 8) * 16, 16)] = v16
        return 0
    lax.fori_loop(0, NT // 16, ci_b, 0)
    _zero16(zb, NSL)
    nb = sid * NSL
    pltpu.sync_copy(zb, cnt_sh.at[pl.ds(nb, NSL)])

    @pl.when(sid == 0)
    def _():
        pltpu.sync_copy(zb.at[pl.ds(0, 128)], p0_sh)
        pltpu.sync_copy(zb.at[pl.ds(0, 128)], p1_sh)
        pltpu.sync_copy(zb.at[pl.ds(0, 128)], pc_sh)

    def ones_b(t, _):
        onev[pl.ds(t * 16, 16)] = jnp.full((16,), 1.0, f32)
        return 0
    lax.fori_loop(0, NT // 16, ones_b, 0)

    def comp(t, _):
        j = t // 8
        c = t % 8
        s16 = s2v[j, pl.ds(c * 16, 16)]
        d16 = d2v[j, pl.ds(c * 16, 16)]
        cs = plsc.load_gather(cidv, [s16])
        ct = plsc.load_gather(cidv, [d16])
        key = jnp.minimum(cs * K0 + ct, NP - 1)
        keyv[j, pl.ds(c * 16, 16)] = key
        valv[pl.ds(t * 16, 16)] = jnp.where(cs != ct, 1.0, 0.0).astype(f32)
        return 0
    lax.fori_loop(0, ET // 16, comp, 0)
    plsc.subcore_barrier()

    def scat(j, _):
        pltpu.sync_copy(valv.at[pl.ds(j * 128, 128)], cnt_sh.at[keyv.at[j]],
                        add=True)
        return 0
    lax.fori_loop(0, ECH, scat, 0)

    def pool(r, _):
        pltpu.sync_copy(c0v.at[pl.ds(r * 128, 128)], p0_sh.at[ci2v.at[r]],
                        add=True)
        pltpu.sync_copy(c1v.at[pl.ds(r * 128, 128)], p1_sh.at[ci2v.at[r]],
                        add=True)
        pltpu.sync_copy(onev.at[pl.ds(r * 128, 128)], pc_sh.at[ci2v.at[r]],
                        add=True)
        return 0
    lax.fori_loop(0, NT // 128, pool, 0)
    plsc.subcore_barrier()
    pltpu.sync_copy(cnt_sh.at[pl.ds(nb, NSL)], zb)
    pltpu.sync_copy(zb, cnt_o.at[cid, pl.ds(nb, NSL)])

    @pl.when(sid == 0)
    def _():
        pltpu.sync_copy(p0_sh, zb.at[pl.ds(0, 128)])
        pltpu.sync_copy(p1_sh, zb.at[pl.ds(128, 128)])
        pltpu.sync_copy(pc_sh, zb.at[pl.ds(256, 128)])
        pltpu.sync_copy(zb.at[pl.ds(0, 384)],
                        pool_o.at[pl.ds(pl.multiple_of(cid * 384, 128), 384)])


@functools.cache
def _pass_f():
  return pl.kernel(
    _pf_body,
    out_type=(jax.ShapeDtypeStruct((2, NP), f32),
              jax.ShapeDtypeStruct((768,), f32)),
    mesh=_mesh(),
    compiler_params=pltpu.CompilerParams(needs_layout_passes=False),
    scratch_types=[
        pltpu.VMEM((ECH, 128), i32), pltpu.VMEM((ECH, 128), i32),
        pltpu.VMEM((NP,), i32), pltpu.VMEM((NT // 128, 128), i32),
        pltpu.VMEM((NT,), f32), pltpu.VMEM((NT,), f32), pltpu.VMEM((NT,), f32),
        pltpu.VMEM((ECH, 128), i32), pltpu.VMEM((ET,), f32),
        pltpu.VMEM((NSL,), f32),
        pltpu.VMEM_SHARED((NP,), f32), pltpu.VMEM_SHARED((128,), f32),
        pltpu.VMEM_SHARED((128,), f32), pltpu.VMEM_SHARED((128,), f32),
    ],
)


def _prep_math(x, W, asv, adv, cb):
    h = lax.dot(x, W, preferred_element_type=f32)
    a_s = jnp.sum(h * asv, axis=1, keepdims=True)
    a_d = jnp.sum(h * adv, axis=1, keepdims=True)
    nr = jnp.sqrt(jnp.sum(cb * cb, axis=1, keepdims=True))
    z = cb / jnp.maximum(nr, 1e-12)
    return h, a_s, a_d, z


def _prep_body(x_ref, W_ref, as_ref, ad_ref, co_ref,
               h_ref, aso_ref, ado_ref, z_ref):
    h, a_s, a_d, z = _prep_math(x_ref[...], W_ref[...], as_ref[...],
                                ad_ref[...], co_ref[...])
    h_ref[...] = h
    aso_ref[...] = a_s
    ado_ref[...] = a_d
    z_ref[...] = z


def _prep(xp, coordp, W, a_src, a_dst):
    return pl.pallas_call(
        _prep_body,
        grid=(GRID,),
        in_specs=[
            pl.BlockSpec((BR, D), lambda i: (i, 0)),
            pl.BlockSpec((D, D), lambda i: (0, 0)),
            pl.BlockSpec((1, D), lambda i: (0, 0)),
            pl.BlockSpec((1, D), lambda i: (0, 0)),
            pl.BlockSpec((BR, 2), lambda i: (i, 0)),
        ],
        out_specs=[
            pl.BlockSpec((BR, D), lambda i: (i, 0)),
            pl.BlockSpec((BR, 1), lambda i: (i, 0)),
            pl.BlockSpec((BR, 1), lambda i: (i, 0)),
            pl.BlockSpec((BR, 2), lambda i: (i, 0)),
        ],
        out_shape=[
            jax.ShapeDtypeStruct((NP, D), f32),
            jax.ShapeDtypeStruct((NP, 1), f32),
            jax.ShapeDtypeStruct((NP, 1), f32),
            jax.ShapeDtypeStruct((NP, 2), f32),
        ],
    )(xp, W, a_src, a_dst, coordp)


def _x1_math(o, s0, s1, a_s, a_d, h0, b, hist, rowid):
    a = a_s + a_d
    a = jnp.where(a > 0, a, 0.2 * a)
    exs = jnp.exp(a)
    rv = jnp.where(rowid < N, 1.0, 0.0)
    x1 = jnp.maximum((o + exs * h0) / (s0 + s1 + exs + 1e-16) + b, 0.0)
    x1 = x1 * rv
    dis = lax.rsqrt(hist + 1.0)
    return x1, dis


def _x1_body(oL0_r, oL1_r, oR0_r, oR1_r, s0_ref, s1_ref, as_ref, ad_ref,
             h0_ref, b_ref, h0c_ref, h1c_ref, x1_ref, dis_ref):
    pid = pl.program_id(0)
    rowid = pid * BR + _fiota((BR, 1), 0)
    o = jnp.concatenate([oL0_r[...] + oL1_r[...], oR0_r[...] + oR1_r[...]],
                        axis=1)
    x1, dis = _x1_math(o, s0_ref[...], s1_ref[...],
                       as_ref[...], ad_ref[...], h0_ref[...], b_ref[...],
                       h0c_ref[...] + h1c_ref[...], rowid)
    x1_ref[...] = x1
    dis_ref[...] = dis


def _x1(oL0, oL1, oR0, oR1, s0, s1, a_s, a_d, h0, b, h0c, h1c):
    col = pl.BlockSpec((BR, 1), lambda i: (i, 0))
    mat = pl.BlockSpec((BR, D), lambda i: (i, 0))
    m64 = pl.BlockSpec((BR, 64), lambda i: (i, 0))
    return pl.pallas_call(
        _x1_body,
        grid=(GRID,),
        in_specs=[m64, m64, m64, m64, col, col, col, col, mat,
                  pl.BlockSpec((1, D), lambda i: (0, 0)), col, col],
        out_specs=[mat, col],
        out_shape=[jax.ShapeDtypeStruct((NP, D), f32),
                   jax.ShapeDtypeStruct((NP, 1), f32)],
    )(oL0, oL1, oR0, oR1, s0, s1, a_s, a_d, h0, b, h0c, h1c)


def _z2_math(a00, a01, a10, a11, z, dis, W1, b1, g1, be1):
    agg0 = a00 + a01
    agg1 = a10 + a11
    aggm = jnp.concatenate([agg0, agg1], axis=1)
    total = aggm + dis * dis * z
    g = lax.dot(total, W1, preferred_element_type=f32) + b1
    return jnp.maximum(g * BNI * g1 + be1, 0.0)


def _z2_body(a00_r, a01_r, a10_r, a11_r, z_r, dis_r, W1_r, b1_r, g1_r, be1_r,
             z2_r):
    z2_r[...] = _z2_math(a00_r[...], a01_r[...], a10_r[...], a11_r[...],
                         z_r[...], dis_r[...], W1_r[...], b1_r[...],
                         g1_r[...], be1_r[...])


def _z2(a00, a01, a10, a11, z, dis, W1p, b1p, g1p, be1p):
    col = pl.BlockSpec((BR, 1), lambda i: (i, 0))
    vec = pl.BlockSpec((1, 64), lambda i: (0, 0))
    return pl.pallas_call(
        _z2_body,
        grid=(GRID,),
        in_specs=[col, col, col, col,
                  pl.BlockSpec((BR, 2), lambda i: (i, 0)), col,
                  pl.BlockSpec((2, 64), lambda i: (0, 0)), vec, vec, vec],
        out_specs=[pl.BlockSpec((BR, 64), lambda i: (i, 0))],
        out_shape=[jax.ShapeDtypeStruct((NP, 64), f32)],
    )(a00, a01, a10, a11, z, dis, W1p, b1p, g1p, be1p)[0]


def _pr_math(a0, a1, z2, dis, W2, b2, g2, be2, Wo, bo, rowid):
    g = lax.dot(a0 + a1 + dis * dis * z2, W2, preferred_element_type=f32) + b2
    z2b = jnp.maximum(g * BNI * g2 + be2, 0.0)
    logits = lax.dot(z2b, Wo, preferred_element_type=f32) + bo
    civ = _fiota((1, D), 1)
    logits = jnp.where(civ < K0, logits, -1e30)
    rmax = jnp.max(logits, axis=1, keepdims=True)
    p = jnp.exp(logits - rmax)
    probs = p / jnp.sum(p, axis=1, keepdims=True)
    rv = rowid < N
    probs = probs * jnp.where(rv, 1.0, 0.0)
    cif = _fiota(logits.shape, 1)
    am = jnp.min(jnp.where(logits == rmax, cif, 1e9), axis=1, keepdims=True)
    cidx = jnp.where(rv, am.astype(i32), NP - NT)
    return probs, cidx


def _pr_body(a0_r, a1_r, z2_r, dis_r, W2_r, b2_r, g2_r, be2_r, Wo_r, bo_r,
             pr_ref, ci_ref):
    pid = pl.program_id(0)
    rowid = pid * BR + _fiota((BR, 1), 0)
    probs, cidx = _pr_math(a0_r[...], a1_r[...], z2_r[...], dis_r[...],
                           W2_r[...], b2_r[...], g2_r[...], be2_r[...],
                           Wo_r[...], bo_r[...], rowid)
    pr_ref[...] = probs
    ci_ref[...] = cidx


def _pr(a0, a1, z2, dis, W2p, b2p, g2p, be2p, Wop, bop):
    col = pl.BlockSpec((BR, 1), lambda i: (i, 0))
    m64 = pl.BlockSpec((BR, 64), lambda i: (i, 0))
    vec = pl.BlockSpec((1, 64), lambda i: (0, 0))
    return pl.pallas_call(
        _pr_body,
        grid=(GRID,),
        in_specs=[m64, m64, m64, col,
                  pl.BlockSpec((64, 64), lambda i: (0, 0)), vec, vec, vec,
                  pl.BlockSpec((64, D), lambda i: (0, 0)),
                  pl.BlockSpec((1, D), lambda i: (0, 0))],
        out_specs=[pl.BlockSpec((BR, D), lambda i: (i, 0)), col],
        out_shape=[jax.ShapeDtypeStruct((NP, D), f32),
                   jax.ShapeDtypeStruct((NP, 1), i32)],
    )(a0, a1, z2, dis, W2p, b2p, g2p, be2p, Wop, bop)


def _nx_body(pr_ref, x1_ref, nx_ref):
    @pl.when(pl.program_id(0) == 0)
    def _():
        nx_ref[...] = jnp.zeros((D, D), f32)
    nx_ref[...] += lax.dot_general(pr_ref[...], x1_ref[...],
                                   (((0,), (0,)), ((), ())),
                                   preferred_element_type=f32)


def _nx(probs, x1):
    return pl.pallas_call(
        _nx_body,
        grid=(GRID,),
        in_specs=[pl.BlockSpec((BR, D), lambda i: (i, 0)),
                  pl.BlockSpec((BR, D), lambda i: (i, 0))],
        out_specs=[pl.BlockSpec((D, D), lambda i: (0, 0))],
        out_shape=[jax.ShapeDtypeStruct((D, D), f32)],
    )(probs, x1)[0]


def _gat_dense(x, mask, W, asv, adv, b, M):
    ri = _fiota((D, D), 0)
    ci = _fiota((D, D), 1)
    h = lax.dot(x, W, preferred_element_type=f32)
    a_col = lax.dot_general(h, asv, (((1,), (1,)), ((), ())),
                            preferred_element_type=f32)
    a_row = lax.dot_general(adv, h, (((1,), (1,)), ((), ())),
                            preferred_element_type=f32)
    e = a_col + a_row
    e = jnp.where(e > 0, e, 0.2 * e)
    eye = jnp.where((ri == ci) & (ci < M), 1.0, 0.0)
    cand = mask + eye
    em = jnp.where(cand > 0, e, -1e30)
    amax = jnp.max(em, axis=0, keepdims=True)
    Wadj = jnp.exp(em - amax)
    ones_col = jnp.ones((D, 1), f32)
    S_col = lax.dot_general(Wadj, ones_col, (((0,), (0,)), ((), ())),
                            preferred_element_type=f32)
    num = lax.dot_general(Wadj, h, (((0,), (0,)), ((), ())),
                          preferred_element_type=f32)
    return num / (S_col + 1e-16) + b


def _dense_math(x, maskraw, curA, curB, W, asv, adv, b,
                W1, b1, g1, be1, W2, b2, g2, be2, Wo, bo, M, K, pool):
    ri = _fiota((D, D), 0)
    ci = _fiota((D, D), 1)
    riv = _fiota((D, 1), 0)
    civ = _fiota((1, D), 1)
    mask = jnp.where((maskraw > 0) & (ri != ci) & (ri < M) & (ci < M), 1.0, 0.0)
    if pool:
        cur = jnp.where(curB > 0, curA / jnp.maximum(curB, 1.0), 0.0)
    else:
        cur = curA
    rv = jnp.where(riv < M, 1.0, 0.0)
    xg = jnp.maximum(_gat_dense(x, mask, W, asv, adv, b, M), 0.0) * rv
    nr = jnp.sqrt(jnp.sum(cur * cur, axis=1, keepdims=True))
    z = cur / jnp.maximum(nr, 1e-12)
    row_any = jnp.max(mask, axis=1, keepdims=True)
    col_any = jnp.max(mask, axis=0, keepdims=True)
    mn0 = jnp.min(jnp.where(row_any > 0, riv, 1e9))
    mn1 = jnp.min(jnp.where(col_any > 0, civ, 1e9))
    P0 = jnp.where(ci == ri + mn0, 1.0, 0.0)
    P1t = jnp.where(ri == ci + mn1, 1.0, 0.0)
    G = lax.dot(P0, lax.dot(mask, P1t, preferred_element_type=f32),
                preferred_element_type=f32)
    ones_col = jnp.ones((D, 1), f32)
    degc = lax.dot_general(G, ones_col, (((0,), (0,)), ((), ())),
                           preferred_element_type=f32) + 1.0
    dis = lax.rsqrt(degc)

    def gcn(hh, b_r):
        t1 = lax.dot_general(G, dis * hh, (((0,), (0,)), ((), ())),
                             preferred_element_type=f32)
        return dis * t1 + dis * dis * hh + b_r

    h1 = lax.dot(z, W1, preferred_element_type=f32)
    z2 = jnp.maximum(gcn(h1, b1) * BNI * g1 + be1, 0.0)
    h2 = lax.dot(z2, W2, preferred_element_type=f32)
    z2b = jnp.maximum(gcn(h2, b2) * BNI * g2 + be2, 0.0)
    logits = lax.dot(z2b, Wo, preferred_element_type=f32) + bo
    logits = jnp.where(civ < K, logits, -1e30)
    rmax = jnp.max(logits, axis=1, keepdims=True)
    p = jnp.exp(logits - rmax)
    probs = p / jnp.sum(p, axis=1, keepdims=True) * rv
    am = jnp.min(jnp.where(logits == rmax, ci, 1e9), axis=1, keepdims=True)
    O = jnp.where((ci == am) & (riv < M), 1.0, 0.0)
    t2 = lax.dot(mask, O, preferred_element_type=f32)
    E2 = lax.dot_general(O, t2, (((0,), (0,)), ((), ())),
                         preferred_element_type=f32)
    mo = jnp.where((E2 > 0) & (ri != ci) & (ri < K) & (ci < K), 1.0, 0.0)
    xo = lax.dot_general(probs, xg, (((0,), (0,)), ((), ())),
                         preferred_element_type=f32)
    sums_p = lax.dot_general(O, cur, (((0,), (0,)), ((), ())),
                             preferred_element_type=f32)
    cntn = lax.dot_general(O, ones_col, (((0,), (0,)), ((), ())),
                           preferred_element_type=f32)
    co = jnp.where(cntn > 0, sums_p / jnp.maximum(cntn, 1.0), 0.0)
    return xo, mo, co


def _dense_body(x_r, m_r, cA_r, cB_r, W_r, as_r, ad_r, b_r,
                W1_r, b1_r, g1_r, be1_r, W2_r, b2_r, g2_r, be2_r,
                Wo_r, bo_r, xo_r, mo_r, co_r, *, M, K, pool):
    xo, mo, co = _dense_math(
        x_r[...], m_r[...], cA_r[...], cB_r[...], W_r[...], as_r[...],
        ad_r[...], b_r[...], W1_r[...], b1_r[...], g1_r[...], be1_r[...],
        W2_r[...], b2_r[...], g2_r[...], be2_r[...], Wo_r[...], bo_r[...],
        M, K, pool)
    xo_r[...] = xo
    mo_r[...] = mo
    co_r[...] = co


def _dense_level(x, mask, curA, curB, conv, pred, M, K, pool):
    full = pl.BlockSpec((D, D), lambda: (0, 0))
    vec = pl.BlockSpec((1, D), lambda: (0, 0))
    colb = pl.BlockSpec((D, 1), lambda: (0, 0))
    Wp = conv['W']
    asv = conv['a_src'][None, :]
    adv = conv['a_dst'][None, :]
    bv = conv['b'][None, :]
    h = pred['W1'].shape[1]
    W1p = jnp.pad(pred['W1'], ((0, D - 2), (0, D - h)))
    W2p = jnp.pad(pred['W2'], ((0, D - h), (0, D - h)))
    Wop = jnp.pad(pred['Wo'], ((0, D - h), (0, D - K)))
    b1p = jnp.pad(pred['b1'], (0, D - h))[None, :]
    g1p = jnp.pad(pred['g1'], (0, D - h))[None, :]
    be1p = jnp.pad(pred['be1'], (0, D - h))[None, :]
    b2p = jnp.pad(pred['b2'], (0, D - h))[None, :]
    g2p = jnp.pad(pred['g2'], (0, D - h))[None, :]
    be2p = jnp.pad(pred['be2'], (0, D - h))[None, :]
    bop = jnp.pad(pred['bo'], (0, D - K))[None, :]
    return pl.pallas_call(
        functools.partial(_dense_body, M=M, K=K, pool=pool),
        in_specs=[full, full, full, colb if pool else full,
                  full, vec, vec, vec,
                  full, vec, vec, vec, full, vec, vec, vec, full, vec],
        out_specs=[full, full, full],
        out_shape=[jax.ShapeDtypeStruct((D, D), f32),
                   jax.ShapeDtypeStruct((D, D), f32),
                   jax.ShapeDtypeStruct((D, D), f32)],
    )(x, mask, curA, curB, Wp, asv, adv, bv, W1p, b1p, g1p, be1p,
      W2p, b2p, g2p, be2p, Wop, bop)


def _final_math(x, maskraw, W, asv, adv, b, M):
    ri = _fiota((D, D), 0)
    ci = _fiota((D, D), 1)
    civ = _fiota((1, D), 1)
    mask = jnp.where((maskraw > 0) & (ri != ci) & (ri < M) & (ci < M), 1.0, 0.0)
    gat = _gat_dense(x, mask, W, asv, adv, b, M)
    rvr = jnp.where(civ < M, 1.0, 0.0)
    return lax.dot_general(rvr, gat, (((1,), (0,)), ((), ())),
                           preferred_element_type=f32) / M


def _final_body(x_r, m_r, W_r, as_r, ad_r, b_r, o_r, *, M):
    o_r[...] = _final_math(x_r[...], m_r[...], W_r[...], as_r[...],
                           ad_r[...], b_r[...], M)


def _final_level(x, mask, conv, M):
    full = pl.BlockSpec((D, D), lambda: (0, 0))
    vec = pl.BlockSpec((1, D), lambda: (0, 0))
    return pl.pallas_call(
        functools.partial(_final_body, M=M),
        in_specs=[full, full, full, vec, vec, vec],
        out_specs=[pl.BlockSpec((1, D), lambda: (0, 0))],
        out_shape=[jax.ShapeDtypeStruct((1, D), f32)],
    )(x, mask, conv['W'], conv['a_src'][None, :], conv['a_dst'][None, :],
      conv['b'][None, :])[0]


def kernel(x, edge_index, batch, coord, params):
    xp = jnp.pad(x, ((0, NP - N), (0, 0)))
    coordp = jnp.pad(coord, ((0, NP - N), (0, 0)))
    src = edge_index[0].astype(i32)
    dst = edge_index[1].astype(i32)
    mn = jnp.min(edge_index, axis=1).astype(i32)
    dump = (N + 2000 + (jnp.arange(EP - E, dtype=i32) % (NP - N - 2000)))

    def pad_e(a):
        return jnp.concatenate([a, dump]).reshape(ERW, 128)

    src2 = pad_e(src)
    dst2 = pad_e(dst)
    srcs2 = pad_e(src - mn[0])
    dsts2 = pad_e(dst - mn[1])

    p0 = params['conv0']
    h0, a_s, a_d, z = _prep(xp, coordp, p0['W'], p0['a_src'][None, :],
                            p0['a_dst'][None, :])

    ex, s2, hist2 = _pass_a()(src2, dst2, dsts2, a_s.reshape(NP),
                            a_d.reshape(NP))
    oL = _pass_rows()(src2, dst2, ex, h0[:, :64])
    oR = _pass_rows()(src2, dst2, ex, h0[:, 64:])
    x1, dis = _x1(oL[0], oL[1], oR[0], oR[1],
                  s2[0].reshape(NP, 1), s2[1].reshape(NP, 1),
                  a_s, a_d, h0, p0['b'][None, :],
                  hist2[0].reshape(NP, 1), hist2[1].reshape(NP, 1))

    nrm, agg = _pass_d()(srcs2, dsts2, dis.reshape(NP), z[:, 0], z[:, 1])

    pr0 = params['pred0']
    h = pr0['W1'].shape[1]
    W1p = jnp.pad(pr0['W1'], ((0, 0), (0, 64 - h)))
    z2 = _z2(agg[0, 0].reshape(NP, 1), agg[1, 0].reshape(NP, 1),
             agg[0, 1].reshape(NP, 1), agg[1, 1].reshape(NP, 1),
             z, dis, W1p, jnp.pad(pr0['b1'], (0, 64 - h))[None, :],
             jnp.pad(pr0['g1'], (0, 64 - h))[None, :],
             jnp.pad(pr0['be1'], (0, 64 - h))[None, :])

    agg2 = _pass_rows()(srcs2, dsts2, nrm, z2)

    W2p = jnp.pad(pr0['W2'], ((0, 64 - h), (0, 64 - h)))
    Wop = jnp.pad(pr0['Wo'], ((0, 64 - h), (0, D - K0)))
    probs, cidx = _pr(agg2[0], agg2[1], z2, dis, W2p,
                      jnp.pad(pr0['b2'], (0, 64 - h))[None, :],
                      jnp.pad(pr0['g2'], (0, 64 - h))[None, :],
                      jnp.pad(pr0['be2'], (0, 64 - h))[None, :],
                      Wop, jnp.pad(pr0['bo'], (0, D - K0))[None, :])

    cidx_f = cidx.reshape(NP)
    cnt, pool = _pass_f()(src2, dst2, cidx_f, coordp[:, 0], coordp[:, 1])
    nx = _nx(probs, x1)

    cnt_t = cnt[0] + cnt[1]
    mask1 = jnp.pad(cnt_t[:K0 * K0].reshape(K0, K0),
                    ((0, D - K0), (0, D - K0)))
    pool_r = pool.reshape(2, 3, 128)
    pool_t = pool_r[0] + pool_r[1]
    sums = jnp.pad(pool_t[0:2].T, ((0, 0), (0, D - 2)))
    cnt_p = pool_t[2].reshape(D, 1)

    x2, mask2, cur2 = _dense_level(nx, mask1, sums, cnt_p,
                                   params['conv1'], params['pred1'],
                                   M=100, K=50, pool=True)
    x3, mask3, cur3 = _dense_level(x2, mask2, cur2, cur2,
                                   params['conv2'], params['pred2'],
                                   M=50, K=10, pool=False)
    return _final_level(x3, mask3, params['conv3'], M=10)


# speedup vs baseline: 34.2323x; 1.5045x over previous
import functools

import jax
import jax.numpy as jnp
from jax import lax
from jax.experimental import pallas as pl
from jax.experimental.pallas import tpu as pltpu
from jax.experimental.pallas import tpu_sc as plsc

N = 10000
E = 320000
D = 128
NP = 12288
EP = 327680
NW = 32
ET = EP // NW
ECH = ET // 128
ERW = EP // 128
NSL = NP // 16
NT = NP // NW
K0 = 100
BR = 1024
GRID = NP // BR
BNI = float(1.0 / (1.0 + 1e-5) ** 0.5)

f32 = jnp.float32
i32 = jnp.int32

@functools.cache
def _mesh():
    return plsc.VectorSubcoreMesh(core_axis_name="c", subcore_axis_name="s",
                                  num_cores=2, num_subcores=16)


def _fiota(shape, dim):
    return lax.broadcasted_iota(i32, shape, dim).astype(f32)


def _wid():
    return lax.axis_index("s") * 2 + lax.axis_index("c")


def _zero16(ref, n):
    def b(t, _):
        ref[pl.ds(t * 16, 16)] = jnp.zeros((16,), f32)
        return 0
    lax.fori_loop(0, n // 16, b, 0)


def _zero2d(ref, rows, cols):
    def b(t, _):
        r = t // (cols // 16)
        c = t % (cols // 16)
        ref[r, pl.ds(c * 16, 16)] = jnp.zeros((16,), f32)
        return 0
    lax.fori_loop(0, rows * (cols // 16), b, 0)


def _pa_body(s2_h, d2_h, ds2_h, asrc_h, adst_h,
             ex_o, s_o, hist_o,
             s2v, d2v, ds2v, asv, adv, exv, onesv, zb, s_sh, h_sh):
    cid = lax.axis_index("c")
    sid = lax.axis_index("s")
    wid = _wid()
    cb = wid * ECH
    pltpu.sync_copy(s2_h.at[pl.ds(cb, ECH)], s2v)
    pltpu.sync_copy(d2_h.at[pl.ds(cb, ECH)], d2v)
    pltpu.sync_copy(ds2_h.at[pl.ds(cb, ECH)], ds2v)
    pltpu.sync_copy(asrc_h, asv)
    pltpu.sync_copy(adst_h, adv)
    _zero16(zb, NSL)
    nb = sid * NSL
    pltpu.sync_copy(zb, s_sh.at[pl.ds(nb, NSL)])
    pltpu.sync_copy(zb, h_sh.at[pl.ds(nb, NSL)])
    _zero16(onesv, ET)

    def ones_b(t, _):
        onesv[pl.ds(t * 16, 16)] = jnp.full((16,), 1.0, f32)
        return 0
    lax.fori_loop(0, ET // 16, ones_b, 0)

    def comp(t, _):
        j = t // 8
        c = t % 8
        s16 = s2v[j, pl.ds(c * 16, 16)]
        d16 = d2v[j, pl.ds(c * 16, 16)]
        a = plsc.load_gather(asv, [s16]) + plsc.load_gather(adv, [d16])
        a = jnp.where(a > 0, a, 0.2 * a)
        exv[pl.ds(t * 16, 16)] = jnp.exp(a)
        return 0
    lax.fori_loop(0, ET // 16, comp, 0)
    plsc.subcore_barrier()

    def scat(j, _):
        pltpu.sync_copy(exv.at[pl.ds(j * 128, 128)], s_sh.at[d2v.at[j]], add=True)
        pltpu.sync_copy(onesv.at[pl.ds(j * 128, 128)], h_sh.at[ds2v.at[j]],
                        add=True)
        return 0
    lax.fori_loop(0, ECH, scat, 0)
    pltpu.sync_copy(exv, ex_o.at[pl.ds(wid * ET, ET)])
    plsc.subcore_barrier()
    pltpu.sync_copy(s_sh.at[pl.ds(nb, NSL)], zb)
    pltpu.sync_copy(zb, s_o.at[cid, pl.ds(nb, NSL)])
    pltpu.sync_copy(h_sh.at[pl.ds(nb, NSL)], zb)
    pltpu.sync_copy(zb, hist_o.at[cid, pl.ds(nb, NSL)])


@functools.cache
def _pass_a():
  return pl.kernel(
    _pa_body,
    out_type=(jax.ShapeDtypeStruct((EP,), f32),
              jax.ShapeDtypeStruct((2, NP), f32),
              jax.ShapeDtypeStruct((2, NP), f32)),
    mesh=_mesh(),
    compiler_params=pltpu.CompilerParams(needs_layout_passes=False),
    scratch_types=[
        pltpu.VMEM((ECH, 128), i32), pltpu.VMEM((ECH, 128), i32),
        pltpu.VMEM((ECH, 128), i32),
        pltpu.VMEM((NP,), f32), pltpu.VMEM((NP,), f32),
        pltpu.VMEM((ET,), f32), pltpu.VMEM((ET,), f32),
        pltpu.VMEM((NSL,), f32),
        pltpu.VMEM_SHARED((NP,), f32), pltpu.VMEM_SHARED((NP,), f32),
    ],
)


def _rows_body(s2_h, d2_h, w_h, tab_h, o_o,
               s2v, d2v, wv, r0, r1, r2, r3, zb, a_sh,
               g0, g1, g2, g3, t0, t1, t2, t3):
    cid = lax.axis_index("c")
    sid = lax.axis_index("s")
    wid = _wid()
    cb = pl.multiple_of(wid * ECH, 8)
    pltpu.sync_copy(s2_h.at[pl.ds(cb, ECH)], s2v)
    pltpu.sync_copy(d2_h.at[pl.ds(cb, ECH)], d2v)
    pltpu.sync_copy(w_h.at[pl.ds(pl.multiple_of(wid * ET, 128), ET)], wv)
    _zero2d(zb, 128, 64)
    nb = sid * NSL

    def zrow(r, _):
        pltpu.sync_copy(zb, a_sh.at[pl.ds(pl.multiple_of(nb + r * 128, 128),
                                          128)])
        return 0
    lax.fori_loop(0, NSL // 128, zrow, 0)
    plsc.subcore_barrier()

    bufs = (r0, r1, r2, r3)
    gsem = (g0, g1, g2, g3)
    ssem = (t0, t1, t2, t3)

    def g_start(j, b):
        pltpu.async_copy(tab_h.at[s2v.at[j]], bufs[b], gsem[b])

    def g_wait(j, b):
        pltpu.make_async_copy(tab_h.at[s2v.at[j]], bufs[b], gsem[b]).wait()

    def s_start(j, b):
        pltpu.async_copy(bufs[b], a_sh.at[d2v.at[j]], ssem[b], add=True)

    def s_wait(j, b):
        pltpu.make_async_copy(bufs[b], a_sh.at[d2v.at[j]], ssem[b]).wait()

    for b in range(3):
        g_start(b, b)

    def quad(q, _):
        for b in range(4):
            j = q * 4 + b
            buf = bufs[b]
            g_wait(j, b)

            def rb(r, _2):
                for u in range(2):
                    rr = r * 2 + u
                    eb = plsc.load_gather(
                        wv, [jnp.zeros((16,), i32) + (j * 128 + rr)])
                    for g in range(4):
                        buf[rr, pl.ds(g * 16, 16)] = (
                            buf[rr, pl.ds(g * 16, 16)] * eb)
                return 0
            lax.fori_loop(0, 64, rb, 0)
            s_start(j, b)
            jr = j + 3
            br = (b + 3) % 4

            @pl.when(jr < ECH)
            def _():
                @pl.when(jr >= 4)
                def _():
                    s_wait(jr - 4, br)
                g_start(jr, br)
        return 0
    lax.fori_loop(0, ECH // 4, quad, 0)
    for b in range(4):
        s_wait(ECH - 4 + b, b)
    plsc.subcore_barrier()

    def wb(r, _):
        off = pl.multiple_of(nb + r * 128, 128)
        pltpu.sync_copy(a_sh.at[pl.ds(off, 128)], r0)
        pltpu.sync_copy(r0, o_o.at[cid, pl.ds(off, 128)])
        return 0
    lax.fori_loop(0, NSL // 128, wb, 0)


@functools.cache
def _pass_rows():
  return pl.kernel(
    _rows_body,
    out_type=jax.ShapeDtypeStruct((2, NP, 64), f32),
    mesh=_mesh(),
    compiler_params=pltpu.CompilerParams(needs_layout_passes=False,
                                         use_tc_tiling_on_sc=False),
    scratch_types=[
        pltpu.VMEM((ECH, 128), i32), pltpu.VMEM((ECH, 128), i32),
        pltpu.VMEM((ET,), f32),
        pltpu.VMEM((128, 64), f32), pltpu.VMEM((128, 64), f32),
        pltpu.VMEM((128, 64), f32), pltpu.VMEM((128, 64), f32),
        pltpu.VMEM((128, 64), f32),
        pltpu.VMEM_SHARED((NP, 64), f32),
        pltpu.SemaphoreType.DMA, pltpu.SemaphoreType.DMA,
        pltpu.SemaphoreType.DMA, pltpu.SemaphoreType.DMA,
        pltpu.SemaphoreType.DMA, pltpu.SemaphoreType.DMA,
        pltpu.SemaphoreType.DMA, pltpu.SemaphoreType.DMA,
    ],
)


def _pd_body(s2_h, d2_h, dis_h, z0_h, z1_h,
             nrm_o, agg_o,
             s2v, d2v, disv, z0v, z1v, nv, v0, v1, zb, a0_sh, a1_sh):
    cid = lax.axis_index("c")
    sid = lax.axis_index("s")
    wid = _wid()
    cb = wid * ECH
    pltpu.sync_copy(s2_h.at[pl.ds(cb, ECH)], s2v)
    pltpu.sync_copy(d2_h.at[pl.ds(cb, ECH)], d2v)
    pltpu.sync_copy(dis_h, disv)
    pltpu.sync_copy(z0_h, z0v)
    pltpu.sync_copy(z1_h, z1v)
    _zero16(zb, NSL)
    nb = sid * NSL
    pltpu.sync_copy(zb, a0_sh.at[pl.ds(nb, NSL)])
    pltpu.sync_copy(zb, a1_sh.at[pl.ds(nb, NSL)])

    def comp(t, _):
        j = t // 8
        c = t % 8
        s16 = s2v[j, pl.ds(c * 16, 16)]
        d16 = d2v[j, pl.ds(c * 16, 16)]
        nr = plsc.load_gather(disv, [s16]) * plsc.load_gather(disv, [d16])
        nv[pl.ds(t * 16, 16)] = nr
        v0[pl.ds(t * 16, 16)] = nr * plsc.load_gather(z0v, [s16])
        v1[pl.ds(t * 16, 16)] = nr * plsc.load_gather(z1v, [s16])
        return 0
    lax.fori_loop(0, ET // 16, comp, 0)
    plsc.subcore_barrier()

    def scat(j, _):
        pltpu.sync_copy(v0.at[pl.ds(j * 128, 128)], a0_sh.at[d2v.at[j]], add=True)
        pltpu.sync_copy(v1.at[pl.ds(j * 128, 128)], a1_sh.at[d2v.at[j]], add=True)
        return 0
    lax.fori_loop(0, ECH, scat, 0)
    pltpu.sync_copy(nv, nrm_o.at[pl.ds(wid * ET, ET)])
    plsc.subcore_barrier()
    pltpu.sync_copy(a0_sh.at[pl.ds(nb, NSL)], zb)
    pltpu.sync_copy(zb, agg_o.at[cid, 0, pl.ds(nb, NSL)])
    pltpu.sync_copy(a1_sh.at[pl.ds(nb, NSL)], zb)
    pltpu.sync_copy(zb, agg_o.at[cid, 1, pl.ds(nb, NSL)])


@functools.cache
def _pass_d():
  return pl.kernel(
    _pd_body,
    out_type=(jax.ShapeDtypeStruct((EP,), f32),
              jax.ShapeDtypeStruct((2, 2, NP), f32)),
    mesh=_mesh(),
    compiler_params=pltpu.CompilerParams(needs_layout_passes=False),
    scratch_types=[
        pltpu.VMEM((ECH, 128), i32), pltpu.VMEM((ECH, 128), i32),
        pltpu.VMEM((NP,), f32), pltpu.VMEM((NP,), f32), pltpu.VMEM((NP,), f32),
        pltpu.VMEM((ET,), f32), pltpu.VMEM((ET,), f32), pltpu.VMEM((ET,), f32),
        pltpu.VMEM((NSL,), f32),
        pltpu.VMEM_SHARED((NP,), f32), pltpu.VMEM_SHARED((NP,), f32),
    ],
)


def _pf_body(s2_h, d2_h, cid_h, c0_h, c1_h,
             cnt_o, pool_o,
             s2v, d2v, cidv, ci2v, c0v, c1v, onev, keyv, valv, zb,
             cnt_sh, p0_sh, p1_sh, pc_sh):
    cid = lax.axis_index("c")
    sid = lax.axis_index("s")
    wid = _wid()
    cb = wid * ECH
    pltpu.sync_copy(s2_h.at[pl.ds(cb, ECH)], s2v)
    pltpu.sync_copy(d2_h.at[pl.ds(cb, ECH)], d2v)
    pltpu.sync_copy(cid_h, cidv)
    nt0 = pl.multiple_of(wid * NT, 128)
    pltpu.sync_copy(c0_h.at[pl.ds(nt0, NT)], c0v)
    pltpu.sync_copy(c1_h.at[pl.ds(nt0, NT)], c1v)

    def ci_b(t, _):
        v16 = cidv[pl.ds(pl.multiple_of(nt0 + t * 16, 16), 16)]
        ci2v[t // 8, pl.ds((t % 8) * 16, 16)] = v16
        return 0
    lax.fori_loop(0, NT // 16, ci_b, 0)
    _zero16(zb, NSL)
    nb = sid * NSL
    pltpu.sync_copy(zb, cnt_sh.at[pl.ds(nb, NSL)])

    @pl.when(sid == 0)
    def _():
        pltpu.sync_copy(zb.at[pl.ds(0, 128)], p0_sh)
        pltpu.sync_copy(zb.at[pl.ds(0, 128)], p1_sh)
        pltpu.sync_copy(zb.at[pl.ds(0, 128)], pc_sh)

    def ones_b(t, _):
        onev[pl.ds(t * 16, 16)] = jnp.full((16,), 1.0, f32)
        return 0
    lax.fori_loop(0, NT // 16, ones_b, 0)

    def comp(t, _):
        j = t // 8
        c = t % 8
        s16 = s2v[j, pl.ds(c * 16, 16)]
        d16 = d2v[j, pl.ds(c * 16, 16)]
        cs = plsc.load_gather(cidv, [s16])
        ct = plsc.load_gather(cidv, [d16])
        key = jnp.minimum(cs * K0 + ct, NP - 1)
        keyv[j, pl.ds(c * 16, 16)] = key
        valv[pl.ds(t * 16, 16)] = jnp.where(cs != ct, 1.0, 0.0).astype(f32)
        return 0
    lax.fori_loop(0, ET // 16, comp, 0)
    plsc.subcore_barrier()

    def scat(j, _):
        pltpu.sync_copy(valv.at[pl.ds(j * 128, 128)], cnt_sh.at[keyv.at[j]],
                        add=True)
        return 0
    lax.fori_loop(0, ECH, scat, 0)

    def pool(r, _):
        pltpu.sync_copy(c0v.at[pl.ds(r * 128, 128)], p0_sh.at[ci2v.at[r]],
                        add=True)
        pltpu.sync_copy(c1v.at[pl.ds(r * 128, 128)], p1_sh.at[ci2v.at[r]],
                        add=True)
        pltpu.sync_copy(onev.at[pl.ds(r * 128, 128)], pc_sh.at[ci2v.at[r]],
                        add=True)
        return 0
    lax.fori_loop(0, NT // 128, pool, 0)
    plsc.subcore_barrier()
    pltpu.sync_copy(cnt_sh.at[pl.ds(nb, NSL)], zb)
    pltpu.sync_copy(zb, cnt_o.at[cid, pl.ds(nb, NSL)])

    @pl.when(sid == 0)
    def _():
        pltpu.sync_copy(p0_sh, zb.at[pl.ds(0, 128)])
        pltpu.sync_copy(p1_sh, zb.at[pl.ds(128, 128)])
        pltpu.sync_copy(pc_sh, zb.at[pl.ds(256, 128)])
        pltpu.sync_copy(zb.at[pl.ds(0, 384)],
                        pool_o.at[pl.ds(pl.multiple_of(cid * 384, 128), 384)])


@functools.cache
def _pass_f():
  return pl.kernel(
    _pf_body,
    out_type=(jax.ShapeDtypeStruct((2, NP), f32),
              jax.ShapeDtypeStruct((768,), f32)),
    mesh=_mesh(),
    compiler_params=pltpu.CompilerParams(needs_layout_passes=False),
    scratch_types=[
        pltpu.VMEM((ECH, 128), i32), pltpu.VMEM((ECH, 128), i32),
        pltpu.VMEM((NP,), i32), pltpu.VMEM((NT // 128, 128), i32),
        pltpu.VMEM((NT,), f32), pltpu.VMEM((NT,), f32), pltpu.VMEM((NT,), f32),
        pltpu.VMEM((ECH, 128), i32), pltpu.VMEM((ET,), f32),
        pltpu.VMEM((NSL,), f32),
        pltpu.VMEM_SHARED((NP,), f32), pltpu.VMEM_SHARED((128,), f32),
        pltpu.VMEM_SHARED((128,), f32), pltpu.VMEM_SHARED((128,), f32),
    ],
)


def _prep_math(x, W, asv, adv, cb):
    h = lax.dot(x, W, preferred_element_type=f32)
    a_s = jnp.sum(h * asv, axis=1, keepdims=True)
    a_d = jnp.sum(h * adv, axis=1, keepdims=True)
    nr = jnp.sqrt(jnp.sum(cb * cb, axis=1, keepdims=True))
    z = cb / jnp.maximum(nr, 1e-12)
    return h, a_s, a_d, z


def _prep_body(x_ref, W_ref, as_ref, ad_ref, co_ref,
               h_ref, aso_ref, ado_ref, z_ref):
    h, a_s, a_d, z = _prep_math(x_ref[...], W_ref[...], as_ref[...],
                                ad_ref[...], co_ref[...])
    h_ref[...] = h
    aso_ref[...] = a_s
    ado_ref[...] = a_d
    z_ref[...] = z


def _prep(xp, coordp, W, a_src, a_dst):
    return pl.pallas_call(
        _prep_body,
        grid=(GRID,),
        in_specs=[
            pl.BlockSpec((BR, D), lambda i: (i, 0)),
            pl.BlockSpec((D, D), lambda i: (0, 0)),
            pl.BlockSpec((1, D), lambda i: (0, 0)),
            pl.BlockSpec((1, D), lambda i: (0, 0)),
            pl.BlockSpec((BR, 2), lambda i: (i, 0)),
        ],
        out_specs=[
            pl.BlockSpec((BR, D), lambda i: (i, 0)),
            pl.BlockSpec((BR, 1), lambda i: (i, 0)),
            pl.BlockSpec((BR, 1), lambda i: (i, 0)),
            pl.BlockSpec((BR, 2), lambda i: (i, 0)),
        ],
        out_shape=[
            jax.ShapeDtypeStruct((NP, D), f32),
            jax.ShapeDtypeStruct((NP, 1), f32),
            jax.ShapeDtypeStruct((NP, 1), f32),
            jax.ShapeDtypeStruct((NP, 2), f32),
        ],
    )(xp, W, a_src, a_dst, coordp)


def _x1_math(o, s0, s1, a_s, a_d, h0, b, hist, rowid):
    a = a_s + a_d
    a = jnp.where(a > 0, a, 0.2 * a)
    exs = jnp.exp(a)
    rv = jnp.where(rowid < N, 1.0, 0.0)
    x1 = jnp.maximum((o + exs * h0) / (s0 + s1 + exs + 1e-16) + b, 0.0)
    x1 = x1 * rv
    dis = lax.rsqrt(hist + 1.0)
    return x1, dis


def _x1_body(oL0_r, oL1_r, oR0_r, oR1_r, s0_ref, s1_ref, as_ref, ad_ref,
             h0_ref, b_ref, h0c_ref, h1c_ref, x1_ref, dis_ref):
    pid = pl.program_id(0)
    rowid = pid * BR + _fiota((BR, 1), 0)
    o = jnp.concatenate([oL0_r[...] + oL1_r[...], oR0_r[...] + oR1_r[...]],
                        axis=1)
    x1, dis = _x1_math(o, s0_ref[...], s1_ref[...],
                       as_ref[...], ad_ref[...], h0_ref[...], b_ref[...],
                       h0c_ref[...] + h1c_ref[...], rowid)
    x1_ref[...] = x1
    dis_ref[...] = dis


def _x1(oL0, oL1, oR0, oR1, s0, s1, a_s, a_d, h0, b, h0c, h1c):
    col = pl.BlockSpec((BR, 1), lambda i: (i, 0))
    mat = pl.BlockSpec((BR, D), lambda i: (i, 0))
    m64 = pl.BlockSpec((BR, 64), lambda i: (i, 0))
    return pl.pallas_call(
        _x1_body,
        grid=(GRID,),
        in_specs=[m64, m64, m64, m64, col, col, col, col, mat,
                  pl.BlockSpec((1, D), lambda i: (0, 0)), col, col],
        out_specs=[mat, col],
        out_shape=[jax.ShapeDtypeStruct((NP, D), f32),
                   jax.ShapeDtypeStruct((NP, 1), f32)],
    )(oL0, oL1, oR0, oR1, s0, s1, a_s, a_d, h0, b, h0c, h1c)


def _z2_math(a00, a01, a10, a11, z, dis, W1, b1, g1, be1):
    agg0 = a00 + a01
    agg1 = a10 + a11
    aggm = jnp.concatenate([agg0, agg1], axis=1)
    total = aggm + dis * dis * z
    g = lax.dot(total, W1, preferred_element_type=f32) + b1
    return jnp.maximum(g * BNI * g1 + be1, 0.0)


def _z2_body(a00_r, a01_r, a10_r, a11_r, z_r, dis_r, W1_r, b1_r, g1_r, be1_r,
             z2_r):
    z2_r[...] = _z2_math(a00_r[...], a01_r[...], a10_r[...], a11_r[...],
                         z_r[...], dis_r[...], W1_r[...], b1_r[...],
                         g1_r[...], be1_r[...])


def _z2(a00, a01, a10, a11, z, dis, W1p, b1p, g1p, be1p):
    col = pl.BlockSpec((BR, 1), lambda i: (i, 0))
    vec = pl.BlockSpec((1, 64), lambda i: (0, 0))
    return pl.pallas_call(
        _z2_body,
        grid=(GRID,),
        in_specs=[col, col, col, col,
                  pl.BlockSpec((BR, 2), lambda i: (i, 0)), col,
                  pl.BlockSpec((2, 64), lambda i: (0, 0)), vec, vec, vec],
        out_specs=[pl.BlockSpec((BR, 64), lambda i: (i, 0))],
        out_shape=[jax.ShapeDtypeStruct((NP, 64), f32)],
    )(a00, a01, a10, a11, z, dis, W1p, b1p, g1p, be1p)[0]


def _pr_math(a0, a1, z2, dis, W2, b2, g2, be2, Wo, bo, rowid):
    g = lax.dot(a0 + a1 + dis * dis * z2, W2, preferred_element_type=f32) + b2
    z2b = jnp.maximum(g * BNI * g2 + be2, 0.0)
    logits = lax.dot(z2b, Wo, preferred_element_type=f32) + bo
    civ = _fiota((1, D), 1)
    logits = jnp.where(civ < K0, logits, -1e30)
    rmax = jnp.max(logits, axis=1, keepdims=True)
    p = jnp.exp(logits - rmax)
    probs = p / jnp.sum(p, axis=1, keepdims=True)
    rv = rowid < N
    probs = probs * jnp.where(rv, 1.0, 0.0)
    cif = _fiota(logits.shape, 1)
    am = jnp.min(jnp.where(logits == rmax, cif, 1e9), axis=1, keepdims=True)
    cidx = jnp.where(rv, am.astype(i32), NP - NT)
    return probs, cidx


def _pr_body(a0_r, a1_r, z2_r, dis_r, W2_r, b2_r, g2_r, be2_r, Wo_r, bo_r,
             pr_ref, ci_ref):
    pid = pl.program_id(0)
    rowid = pid * BR + _fiota((BR, 1), 0)
    probs, cidx = _pr_math(a0_r[...], a1_r[...], z2_r[...], dis_r[...],
                           W2_r[...], b2_r[...], g2_r[...], be2_r[...],
                           Wo_r[...], bo_r[...], rowid)
    pr_ref[...] = probs
    ci_ref[...] = cidx


def _pr(a0, a1, z2, dis, W2p, b2p, g2p, be2p, Wop, bop):
    col = pl.BlockSpec((BR, 1), lambda i: (i, 0))
    m64 = pl.BlockSpec((BR, 64), lambda i: (i, 0))
    vec = pl.BlockSpec((1, 64), lambda i: (0, 0))
    return pl.pallas_call(
        _pr_body,
        grid=(GRID,),
        in_specs=[m64, m64, m64, col,
                  pl.BlockSpec((64, 64), lambda i: (0, 0)), vec, vec, vec,
                  pl.BlockSpec((64, D), lambda i: (0, 0)),
                  pl.BlockSpec((1, D), lambda i: (0, 0))],
        out_specs=[pl.BlockSpec((BR, D), lambda i: (i, 0)), col],
        out_shape=[jax.ShapeDtypeStruct((NP, D), f32),
                   jax.ShapeDtypeStruct((NP, 1), i32)],
    )(a0, a1, z2, dis, W2p, b2p, g2p, be2p, Wop, bop)


def _nx_body(pr_ref, x1_ref, nx_ref):
    @pl.when(pl.program_id(0) == 0)
    def _():
        nx_ref[...] = jnp.zeros((D, D), f32)
    nx_ref[...] += lax.dot_general(pr_ref[...], x1_ref[...],
                                   (((0,), (0,)), ((), ())),
                                   preferred_element_type=f32)


def _nx(probs, x1):
    return pl.pallas_call(
        _nx_body,
        grid=(GRID,),
        in_specs=[pl.BlockSpec((BR, D), lambda i: (i, 0)),
                  pl.BlockSpec((BR, D), lambda i: (i, 0))],
        out_specs=[pl.BlockSpec((D, D), lambda i: (0, 0))],
        out_shape=[jax.ShapeDtypeStruct((D, D), f32)],
    )(probs, x1)[0]


def _gat_dense(x, mask, W, asv, adv, b, M):
    ri = _fiota((D, D), 0)
    ci = _fiota((D, D), 1)
    h = lax.dot(x, W, preferred_element_type=f32)
    a_col = lax.dot_general(h, asv, (((1,), (1,)), ((), ())),
                            preferred_element_type=f32)
    a_row = lax.dot_general(adv, h, (((1,), (1,)), ((), ())),
                            preferred_element_type=f32)
    e = a_col + a_row
    e = jnp.where(e > 0, e, 0.2 * e)
    eye = jnp.where((ri == ci) & (ci < M), 1.0, 0.0)
    cand = mask + eye
    em = jnp.where(cand > 0, e, -1e30)
    amax = jnp.max(em, axis=0, keepdims=True)
    Wadj = jnp.exp(em - amax)
    ones_col = jnp.ones((D, 1), f32)
    S_col = lax.dot_general(Wadj, ones_col, (((0,), (0,)), ((), ())),
                            preferred_element_type=f32)
    num = lax.dot_general(Wadj, h, (((0,), (0,)), ((), ())),
                          preferred_element_type=f32)
    return num / (S_col + 1e-16) + b


def _dense_math(x, maskraw, curA, curB, W, asv, adv, b,
                W1, b1, g1, be1, W2, b2, g2, be2, Wo, bo, M, K, pool):
    ri = _fiota((D, D), 0)
    ci = _fiota((D, D), 1)
    riv = _fiota((D, 1), 0)
    civ = _fiota((1, D), 1)
    mask = jnp.where((maskraw > 0) & (ri != ci) & (ri < M) & (ci < M), 1.0, 0.0)
    if pool:
        cur = jnp.where(curB > 0, curA / jnp.maximum(curB, 1.0), 0.0)
    else:
        cur = curA
    rv = jnp.where(riv < M, 1.0, 0.0)
    xg = jnp.maximum(_gat_dense(x, mask, W, asv, adv, b, M), 0.0) * rv
    nr = jnp.sqrt(jnp.sum(cur * cur, axis=1, keepdims=True))
    z = cur / jnp.maximum(nr, 1e-12)
    row_any = jnp.max(mask, axis=1, keepdims=True)
    col_any = jnp.max(mask, axis=0, keepdims=True)
    mn0 = jnp.min(jnp.where(row_any > 0, riv, 1e9))
    mn1 = jnp.min(jnp.where(col_any > 0, civ, 1e9))
    P0 = jnp.where(ci == ri + mn0, 1.0, 0.0)
    P1t = jnp.where(ri == ci + mn1, 1.0, 0.0)
    G = lax.dot(P0, lax.dot(mask, P1t, preferred_element_type=f32),
                preferred_element_type=f32)
    ones_col = jnp.ones((D, 1), f32)
    degc = lax.dot_general(G, ones_col, (((0,), (0,)), ((), ())),
                           preferred_element_type=f32) + 1.0
    dis = lax.rsqrt(degc)

    def gcn(hh, b_r):
        t1 = lax.dot_general(G, dis * hh, (((0,), (0,)), ((), ())),
                             preferred_element_type=f32)
        return dis * t1 + dis * dis * hh + b_r

    h1 = lax.dot(z, W1, preferred_element_type=f32)
    z2 = jnp.maximum(gcn(h1, b1) * BNI * g1 + be1, 0.0)
    h2 = lax.dot(z2, W2, preferred_element_type=f32)
    z2b = jnp.maximum(gcn(h2, b2) * BNI * g2 + be2, 0.0)
    logits = lax.dot(z2b, Wo, preferred_element_type=f32) + bo
    logits = jnp.where(civ < K, logits, -1e30)
    rmax = jnp.max(logits, axis=1, keepdims=True)
    p = jnp.exp(logits - rmax)
    probs = p / jnp.sum(p, axis=1, keepdims=True) * rv
    am = jnp.min(jnp.where(logits == rmax, ci, 1e9), axis=1, keepdims=True)
    O = jnp.where((ci == am) & (riv < M), 1.0, 0.0)
    t2 = lax.dot(mask, O, preferred_element_type=f32)
    E2 = lax.dot_general(O, t2, (((0,), (0,)), ((), ())),
                         preferred_element_type=f32)
    mo = jnp.where((E2 > 0) & (ri != ci) & (ri < K) & (ci < K), 1.0, 0.0)
    xo = lax.dot_general(probs, xg, (((0,), (0,)), ((), ())),
                         preferred_element_type=f32)
    sums_p = lax.dot_general(O, cur, (((0,), (0,)), ((), ())),
                             preferred_element_type=f32)
    cntn = lax.dot_general(O, ones_col, (((0,), (0,)), ((), ())),
                           preferred_element_type=f32)
    co = jnp.where(cntn > 0, sums_p / jnp.maximum(cntn, 1.0), 0.0)
    return xo, mo, co


def _dense_body(x_r, m_r, cA_r, cB_r, W_r, as_r, ad_r, b_r,
                W1_r, b1_r, g1_r, be1_r, W2_r, b2_r, g2_r, be2_r,
                Wo_r, bo_r, xo_r, mo_r, co_r, *, M, K, pool):
    xo, mo, co = _dense_math(
        x_r[...], m_r[...], cA_r[...], cB_r[...], W_r[...], as_r[...],
        ad_r[...], b_r[...], W1_r[...], b1_r[...], g1_r[...], be1_r[...],
        W2_r[...], b2_r[...], g2_r[...], be2_r[...], Wo_r[...], bo_r[...],
        M, K, pool)
    xo_r[...] = xo
    mo_r[...] = mo
    co_r[...] = co


def _dense_level(x, mask, curA, curB, conv, pred, M, K, pool):
    full = pl.BlockSpec((D, D), lambda: (0, 0))
    vec = pl.BlockSpec((1, D), lambda: (0, 0))
    colb = pl.BlockSpec((D, 1), lambda: (0, 0))
    Wp = conv['W']
    asv = conv['a_src'][None, :]
    adv = conv['a_dst'][None, :]
    bv = conv['b'][None, :]
    h = pred['W1'].shape[1]
    W1p = jnp.pad(pred['W1'], ((0, D - 2), (0, D - h)))
    W2p = jnp.pad(pred['W2'], ((0, D - h), (0, D - h)))
    Wop = jnp.pad(pred['Wo'], ((0, D - h), (0, D - K)))
    b1p = jnp.pad(pred['b1'], (0, D - h))[None, :]
    g1p = jnp.pad(pred['g1'], (0, D - h))[None, :]
    be1p = jnp.pad(pred['be1'], (0, D - h))[None, :]
    b2p = jnp.pad(pred['b2'], (0, D - h))[None, :]
    g2p = jnp.pad(pred['g2'], (0, D - h))[None, :]
    be2p = jnp.pad(pred['be2'], (0, D - h))[None, :]
    bop = jnp.pad(pred['bo'], (0, D - K))[None, :]
    return pl.pallas_call(
        functools.partial(_dense_body, M=M, K=K, pool=pool),
        in_specs=[full, full, full, colb if pool else full,
                  full, vec, vec, vec,
                  full, vec, vec, vec, full, vec, vec, vec, full, vec],
        out_specs=[full, full, full],
        out_shape=[jax.ShapeDtypeStruct((D, D), f32),
                   jax.ShapeDtypeStruct((D, D), f32),
                   jax.ShapeDtypeStruct((D, D), f32)],
    )(x, mask, curA, curB, Wp, asv, adv, bv, W1p, b1p, g1p, be1p,
      W2p, b2p, g2p, be2p, Wop, bop)


def _final_math(x, maskraw, W, asv, adv, b, M):
    ri = _fiota((D, D), 0)
    ci = _fiota((D, D), 1)
    civ = _fiota((1, D), 1)
    mask = jnp.where((maskraw > 0) & (ri != ci) & (ri < M) & (ci < M), 1.0, 0.0)
    gat = _gat_dense(x, mask, W, asv, adv, b, M)
    rvr = jnp.where(civ < M, 1.0, 0.0)
    return lax.dot_general(rvr, gat, (((1,), (0,)), ((), ())),
                           preferred_element_type=f32) / M


def _final_body(x_r, m_r, W_r, as_r, ad_r, b_r, o_r, *, M):
    o_r[...] = _final_math(x_r[...], m_r[...], W_r[...], as_r[...],
                           ad_r[...], b_r[...], M)


def _final_level(x, mask, conv, M):
    full = pl.BlockSpec((D, D), lambda: (0, 0))
    vec = pl.BlockSpec((1, D), lambda: (0, 0))
    return pl.pallas_call(
        functools.partial(_final_body, M=M),
        in_specs=[full, full, full, vec, vec, vec],
        out_specs=[pl.BlockSpec((1, D), lambda: (0, 0))],
        out_shape=[jax.ShapeDtypeStruct((1, D), f32)],
    )(x, mask, conv['W'], conv['a_src'][None, :], conv['a_dst'][None, :],
      conv['b'][None, :])[0]


def kernel(x, edge_index, batch, coord, params):
    xp = jnp.pad(x, ((0, NP - N), (0, 0)))
    coordp = jnp.pad(coord, ((0, NP - N), (0, 0)))
    src = edge_index[0].astype(i32)
    dst = edge_index[1].astype(i32)
    mn = jnp.min(edge_index, axis=1).astype(i32)
    dump = (N + 2000 + (jnp.arange(EP - E, dtype=i32) % (NP - N - 2000)))

    def pad_e(a):
        return jnp.concatenate([a, dump]).reshape(ERW, 128)

    src2 = pad_e(src)
    dst2 = pad_e(dst)
    srcs2 = pad_e(src - mn[0])
    dsts2 = pad_e(dst - mn[1])

    p0 = params['conv0']
    h0, a_s, a_d, z = _prep(xp, coordp, p0['W'], p0['a_src'][None, :],
                            p0['a_dst'][None, :])

    ex, s2, hist2 = _pass_a()(src2, dst2, dsts2, a_s.reshape(NP),
                            a_d.reshape(NP))
    oL = _pass_rows()(src2, dst2, ex, h0[:, :64])
    oR = _pass_rows()(src2, dst2, ex, h0[:, 64:])
    x1, dis = _x1(oL[0], oL[1], oR[0], oR[1],
                  s2[0].reshape(NP, 1), s2[1].reshape(NP, 1),
                  a_s, a_d, h0, p0['b'][None, :],
                  hist2[0].reshape(NP, 1), hist2[1].reshape(NP, 1))

    nrm, agg = _pass_d()(srcs2, dsts2, dis.reshape(NP), z[:, 0], z[:, 1])

    pr0 = params['pred0']
    h = pr0['W1'].shape[1]
    W1p = jnp.pad(pr0['W1'], ((0, 0), (0, 64 - h)))
    z2 = _z2(agg[0, 0].reshape(NP, 1), agg[1, 0].reshape(NP, 1),
             agg[0, 1].reshape(NP, 1), agg[1, 1].reshape(NP, 1),
             z, dis, W1p, jnp.pad(pr0['b1'], (0, 64 - h))[None, :],
             jnp.pad(pr0['g1'], (0, 64 - h))[None, :],
             jnp.pad(pr0['be1'], (0, 64 - h))[None, :])

    agg2 = _pass_rows()(srcs2, dsts2, nrm, z2)

    W2p = jnp.pad(pr0['W2'], ((0, 64 - h), (0, 64 - h)))
    Wop = jnp.pad(pr0['Wo'], ((0, 64 - h), (0, D - K0)))
    probs, cidx = _pr(agg2[0], agg2[1], z2, dis, W2p,
                      jnp.pad(pr0['b2'], (0, 64 - h))[None, :],
                      jnp.pad(pr0['g2'], (0, 64 - h))[None, :],
                      jnp.pad(pr0['be2'], (0, 64 - h))[None, :],
                      Wop, jnp.pad(pr0['bo'], (0, D - K0))[None, :])

    cidx_f = cidx.reshape(NP)
    cnt, pool = _pass_f()(src2, dst2, cidx_f, coordp[:, 0], coordp[:, 1])
    nx = _nx(probs, x1)

    cnt_t = cnt[0] + cnt[1]
    mask1 = jnp.pad(cnt_t[:K0 * K0].reshape(K0, K0),
                    ((0, D - K0), (0, D - K0)))
    pool_r = pool.reshape(2, 3, 128)
    pool_t = pool_r[0] + pool_r[1]
    sums = jnp.pad(pool_t[0:2].T, ((0, 0), (0, D - 2)))
    cnt_p = pool_t[2].reshape(D, 1)

    x2, mask2, cur2 = _dense_level(nx, mask1, sums, cnt_p,
                                   params['conv1'], params['pred1'],
                                   M=100, K=50, pool=True)
    x3, mask3, cur3 = _dense_level(x2, mask2, cur2, cur2,
                                   params['conv2'], params['pred2'],
                                   M=50, K=10, pool=False)
    return _final_level(x3, mask3, params['conv3'], M=10)


# speedup vs baseline: 34.7085x; 1.0139x over previous
import functools

import jax
import jax.numpy as jnp
from jax import lax
from jax.experimental import pallas as pl
from jax.experimental.pallas import tpu as pltpu
from jax.experimental.pallas import tpu_sc as plsc

N = 10000
E = 320000
D = 128
NP = 12288
EP = 327680
NW = 32
ET = EP // NW
ECH = ET // 128
ERW = EP // 128
NSL = NP // 16
NT = NP // NW
K0 = 100
BR = 1024
GRID = NP // BR
BNI = float(1.0 / (1.0 + 1e-5) ** 0.5)

f32 = jnp.float32
i32 = jnp.int32

@functools.cache
def _mesh():
    return plsc.VectorSubcoreMesh(core_axis_name="c", subcore_axis_name="s",
                                  num_cores=2, num_subcores=16)


def _fiota(shape, dim):
    return lax.broadcasted_iota(i32, shape, dim).astype(f32)


def _wid():
    return lax.axis_index("s") * 2 + lax.axis_index("c")


def _zero16(ref, n):
    def b(t, _):
        ref[pl.ds(t * 16, 16)] = jnp.zeros((16,), f32)
        return 0
    lax.fori_loop(0, n // 16, b, 0)


def _zero2d(ref, rows, cols):
    def b(t, _):
        r = t // (cols // 16)
        c = t % (cols // 16)
        ref[r, pl.ds(c * 16, 16)] = jnp.zeros((16,), f32)
        return 0
    lax.fori_loop(0, rows * (cols // 16), b, 0)


def _pa_body(s2_h, d2_h, ds2_h, asrc_h, adst_h,
             ex_o, s_o, hist_o,
             s2v, d2v, ds2v, asv, adv, exv, onesv, zb, s_sh, h_sh):
    cid = lax.axis_index("c")
    sid = lax.axis_index("s")
    wid = _wid()
    cb = wid * ECH
    pltpu.sync_copy(s2_h.at[pl.ds(cb, ECH)], s2v)
    pltpu.sync_copy(d2_h.at[pl.ds(cb, ECH)], d2v)
    pltpu.sync_copy(ds2_h.at[pl.ds(cb, ECH)], ds2v)
    pltpu.sync_copy(asrc_h, asv)
    pltpu.sync_copy(adst_h, adv)
    _zero16(zb, NSL)
    nb = sid * NSL
    pltpu.sync_copy(zb, s_sh.at[pl.ds(nb, NSL)])
    pltpu.sync_copy(zb, h_sh.at[pl.ds(nb, NSL)])
    def ones_b(t, _):
        onesv[pl.ds(t * 16, 16)] = jnp.full((16,), 1.0, f32)
        return 0
    lax.fori_loop(0, 8, ones_b, 0)

    def comp(t, _):
        j = t // 8
        c = t % 8
        s16 = s2v[j, pl.ds(c * 16, 16)]
        d16 = d2v[j, pl.ds(c * 16, 16)]
        a = plsc.load_gather(asv, [s16]) + plsc.load_gather(adv, [d16])
        a = jnp.where(a > 0, a, 0.2 * a)
        exv[pl.ds(t * 16, 16)] = jnp.exp(a)
        return 0
    lax.fori_loop(0, ET // 16, comp, 0)
    plsc.subcore_barrier()

    def scat(j, _):
        pltpu.sync_copy(exv.at[pl.ds(j * 128, 128)], s_sh.at[d2v.at[j]], add=True)
        pltpu.sync_copy(onesv, h_sh.at[ds2v.at[j]], add=True)
        return 0
    lax.fori_loop(0, ECH, scat, 0)
    pltpu.sync_copy(exv, ex_o.at[pl.ds(wid * ET, ET)])
    plsc.subcore_barrier()
    pltpu.sync_copy(s_sh.at[pl.ds(nb, NSL)], zb)
    pltpu.sync_copy(zb, s_o.at[cid, pl.ds(nb, NSL)])
    pltpu.sync_copy(h_sh.at[pl.ds(nb, NSL)], zb)
    pltpu.sync_copy(zb, hist_o.at[cid, pl.ds(nb, NSL)])


@functools.cache
def _pass_a():
  return pl.kernel(
    _pa_body,
    out_type=(jax.ShapeDtypeStruct((EP,), f32),
              jax.ShapeDtypeStruct((2, NP), f32),
              jax.ShapeDtypeStruct((2, NP), f32)),
    mesh=_mesh(),
    compiler_params=pltpu.CompilerParams(needs_layout_passes=False),
    scratch_types=[
        pltpu.VMEM((ECH, 128), i32), pltpu.VMEM((ECH, 128), i32),
        pltpu.VMEM((ECH, 128), i32),
        pltpu.VMEM((NP,), f32), pltpu.VMEM((NP,), f32),
        pltpu.VMEM((ET,), f32), pltpu.VMEM((128,), f32),
        pltpu.VMEM((NSL,), f32),
        pltpu.VMEM_SHARED((NP,), f32), pltpu.VMEM_SHARED((NP,), f32),
    ],
)


def _rows_body(s2_h, d2_h, w_h, tab_h, o_o,
               s2v, d2v, wv, r0, r1, r2, r3, zb, a_sh,
               g0, g1, g2, g3, t0, t1, t2, t3):
    cid = lax.axis_index("c")
    sid = lax.axis_index("s")
    wid = _wid()
    cb = pl.multiple_of(wid * ECH, 8)
    pltpu.sync_copy(s2_h.at[pl.ds(cb, ECH)], s2v)
    pltpu.sync_copy(d2_h.at[pl.ds(cb, ECH)], d2v)
    pltpu.sync_copy(w_h.at[pl.ds(pl.multiple_of(wid * ET, 128), ET)], wv)
    _zero2d(zb, 128, 64)
    nb = sid * NSL

    def zrow(r, _):
        pltpu.sync_copy(zb, a_sh.at[pl.ds(pl.multiple_of(nb + r * 128, 128),
                                          128)])
        return 0
    lax.fori_loop(0, NSL // 128, zrow, 0)
    plsc.subcore_barrier()

    bufs = (r0, r1, r2, r3)
    gsem = (g0, g1, g2, g3)
    ssem = (t0, t1, t2, t3)

    def g_start(j, b):
        pltpu.async_copy(tab_h.at[s2v.at[j]], bufs[b], gsem[b])

    def g_wait(j, b):
        pltpu.make_async_copy(tab_h.at[s2v.at[j]], bufs[b], gsem[b]).wait()

    def s_start(j, b):
        pltpu.async_copy(bufs[b], a_sh.at[d2v.at[j]], ssem[b], add=True)

    def s_wait(j, b):
        pltpu.make_async_copy(bufs[b], a_sh.at[d2v.at[j]], ssem[b]).wait()

    for b in range(3):
        g_start(b, b)

    def quad(q, _):
        for b in range(4):
            j = q * 4 + b
            buf = bufs[b]
            g_wait(j, b)

            def rb(r, _2):
                for u in range(4):
                    rr = r * 4 + u
                    eb = plsc.load_gather(
                        wv, [jnp.zeros((16,), i32) + (j * 128 + rr)])
                    for g in range(4):
                        buf[rr, pl.ds(g * 16, 16)] = (
                            buf[rr, pl.ds(g * 16, 16)] * eb)
                return 0
            lax.fori_loop(0, 32, rb, 0)
            s_start(j, b)
            jr = j + 3
            br = (b + 3) % 4

            @pl.when(jr < ECH)
            def _():
                @pl.when(jr >= 4)
                def _():
                    s_wait(jr - 4, br)
                g_start(jr, br)
        return 0
    lax.fori_loop(0, ECH // 4, quad, 0)
    for b in range(4):
        s_wait(ECH - 4 + b, b)
    plsc.subcore_barrier()

    def wb(r, _):
        off = pl.multiple_of(nb + r * 128, 128)
        pltpu.sync_copy(a_sh.at[pl.ds(off, 128)], r0)
        pltpu.sync_copy(r0, o_o.at[cid, pl.ds(off, 128)])
        return 0
    lax.fori_loop(0, NSL // 128, wb, 0)


@functools.cache
def _pass_rows():
  return pl.kernel(
    _rows_body,
    out_type=jax.ShapeDtypeStruct((2, NP, 64), f32),
    mesh=_mesh(),
    compiler_params=pltpu.CompilerParams(needs_layout_passes=False,
                                         use_tc_tiling_on_sc=False),
    scratch_types=[
        pltpu.VMEM((ECH, 128), i32), pltpu.VMEM((ECH, 128), i32),
        pltpu.VMEM((ET,), f32),
        pltpu.VMEM((128, 64), f32), pltpu.VMEM((128, 64), f32),
        pltpu.VMEM((128, 64), f32), pltpu.VMEM((128, 64), f32),
        pltpu.VMEM((128, 64), f32),
        pltpu.VMEM_SHARED((NP, 64), f32),
        pltpu.SemaphoreType.DMA, pltpu.SemaphoreType.DMA,
        pltpu.SemaphoreType.DMA, pltpu.SemaphoreType.DMA,
        pltpu.SemaphoreType.DMA, pltpu.SemaphoreType.DMA,
        pltpu.SemaphoreType.DMA, pltpu.SemaphoreType.DMA,
    ],
)


def _pd_body(s2_h, d2_h, dis_h, z0_h, z1_h,
             nrm_o, agg_o,
             s2v, d2v, disv, z0v, z1v, nv, v0, v1, zb, a0_sh, a1_sh):
    cid = lax.axis_index("c")
    sid = lax.axis_index("s")
    wid = _wid()
    cb = wid * ECH
    pltpu.sync_copy(s2_h.at[pl.ds(cb, ECH)], s2v)
    pltpu.sync_copy(d2_h.at[pl.ds(cb, ECH)], d2v)
    pltpu.sync_copy(dis_h, disv)
    pltpu.sync_copy(z0_h, z0v)
    pltpu.sync_copy(z1_h, z1v)
    _zero16(zb, NSL)
    nb = sid * NSL
    pltpu.sync_copy(zb, a0_sh.at[pl.ds(nb, NSL)])
    pltpu.sync_copy(zb, a1_sh.at[pl.ds(nb, NSL)])

    def comp(t, _):
        j = t // 8
        c = t % 8
        s16 = s2v[j, pl.ds(c * 16, 16)]
        d16 = d2v[j, pl.ds(c * 16, 16)]
        nr = plsc.load_gather(disv, [s16]) * plsc.load_gather(disv, [d16])
        nv[pl.ds(t * 16, 16)] = nr
        v0[pl.ds(t * 16, 16)] = nr * plsc.load_gather(z0v, [s16])
        v1[pl.ds(t * 16, 16)] = nr * plsc.load_gather(z1v, [s16])
        return 0
    lax.fori_loop(0, ET // 16, comp, 0)
    plsc.subcore_barrier()

    def scat(j, _):
        pltpu.sync_copy(v0.at[pl.ds(j * 128, 128)], a0_sh.at[d2v.at[j]], add=True)
        pltpu.sync_copy(v1.at[pl.ds(j * 128, 128)], a1_sh.at[d2v.at[j]], add=True)
        return 0
    lax.fori_loop(0, ECH, scat, 0)
    pltpu.sync_copy(nv, nrm_o.at[pl.ds(wid * ET, ET)])
    plsc.subcore_barrier()
    pltpu.sync_copy(a0_sh.at[pl.ds(nb, NSL)], zb)
    pltpu.sync_copy(zb, agg_o.at[cid, 0, pl.ds(nb, NSL)])
    pltpu.sync_copy(a1_sh.at[pl.ds(nb, NSL)], zb)
    pltpu.sync_copy(zb, agg_o.at[cid, 1, pl.ds(nb, NSL)])


@functools.cache
def _pass_d():
  return pl.kernel(
    _pd_body,
    out_type=(jax.ShapeDtypeStruct((EP,), f32),
              jax.ShapeDtypeStruct((2, 2, NP), f32)),
    mesh=_mesh(),
    compiler_params=pltpu.CompilerParams(needs_layout_passes=False),
    scratch_types=[
        pltpu.VMEM((ECH, 128), i32), pltpu.VMEM((ECH, 128), i32),
        pltpu.VMEM((NP,), f32), pltpu.VMEM((NP,), f32), pltpu.VMEM((NP,), f32),
        pltpu.VMEM((ET,), f32), pltpu.VMEM((ET,), f32), pltpu.VMEM((ET,), f32),
        pltpu.VMEM((NSL,), f32),
        pltpu.VMEM_SHARED((NP,), f32), pltpu.VMEM_SHARED((NP,), f32),
    ],
)


def _pf_body(s2_h, d2_h, cid_h, c0_h, c1_h,
             cnt_o, pool_o,
             s2v, d2v, cidv, ci2v, c0v, c1v, onev, keyv, valv, zb,
             cnt_sh, p0_sh, p1_sh, pc_sh):
    cid = lax.axis_index("c")
    sid = lax.axis_index("s")
    wid = _wid()
    cb = wid * ECH
    pltpu.sync_copy(s2_h.at[pl.ds(cb, ECH)], s2v)
    pltpu.sync_copy(d2_h.at[pl.ds(cb, ECH)], d2v)
    pltpu.sync_copy(cid_h, cidv)
    nt0 = pl.multiple_of(wid * NT, 128)
    pltpu.sync_copy(c0_h.at[pl.ds(nt0, NT)], c0v)
    pltpu.sync_copy(c1_h.at[pl.ds(nt0, NT)], c1v)

    def ci_b(t, _):
        v16 = cidv[pl.ds(pl.multiple_of(nt0 + t * 16, 16), 16)]
        ci2v[t // 8, pl.ds((t % 8) * 16, 16)] = v16
        return 0
    lax.fori_loop(0, NT // 16, ci_b, 0)
    _zero16(zb, NSL)
    nb = sid * NSL
    pltpu.sync_copy(zb, cnt_sh.at[pl.ds(nb, NSL)])

    @pl.when(sid == 0)
    def _():
        pltpu.sync_copy(zb.at[pl.ds(0, 128)], p0_sh)
        pltpu.sync_copy(zb.at[pl.ds(0, 128)], p1_sh)
        pltpu.sync_copy(zb.at[pl.ds(0, 128)], pc_sh)

    def ones_b(t, _):
        onev[pl.ds(t * 16, 16)] = jnp.full((16,), 1.0, f32)
        return 0
    lax.fori_loop(0, NT // 16, ones_b, 0)

    def comp(t, _):
        j = t // 8
        c = t % 8
        s16 = s2v[j, pl.ds(c * 16, 16)]
        d16 = d2v[j, pl.ds(c * 16, 16)]
        cs = plsc.load_gather(cidv, [s16])
        ct = plsc.load_gather(cidv, [d16])
        key = jnp.minimum(cs * K0 + ct, NP - 1)
        keyv[j, pl.ds(c * 16, 16)] = key
        valv[pl.ds(t * 16, 16)] = jnp.where(cs != ct, 1.0, 0.0).astype(f32)
        return 0
    lax.fori_loop(0, ET // 16, comp, 0)
    plsc.subcore_barrier()

    def scat(j, _):
        pltpu.sync_copy(valv.at[pl.ds(j * 128, 128)], cnt_sh.at[keyv.at[j]],
                        add=True)
        return 0
    lax.fori_loop(0, ECH, scat, 0)

    def pool(r, _):
        pltpu.sync_copy(c0v.at[pl.ds(r * 128, 128)], p0_sh.at[ci2v.at[r]],
                        add=True)
        pltpu.sync_copy(c1v.at[pl.ds(r * 128, 128)], p1_sh.at[ci2v.at[r]],
                        add=True)
        pltpu.sync_copy(onev.at[pl.ds(r * 128, 128)], pc_sh.at[ci2v.at[r]],
                        add=True)
        return 0
    lax.fori_loop(0, NT // 128, pool, 0)
    plsc.subcore_barrier()
    pltpu.sync_copy(cnt_sh.at[pl.ds(nb, NSL)], zb)
    pltpu.sync_copy(zb, cnt_o.at[cid, pl.ds(nb, NSL)])

    @pl.when(sid == 0)
    def _():
        pltpu.sync_copy(p0_sh, zb.at[pl.ds(0, 128)])
        pltpu.sync_copy(p1_sh, zb.at[pl.ds(128, 128)])
        pltpu.sync_copy(pc_sh, zb.at[pl.ds(256, 128)])
        pltpu.sync_copy(zb.at[pl.ds(0, 384)],
                        pool_o.at[pl.ds(pl.multiple_of(cid * 384, 128), 384)])


@functools.cache
def _pass_f():
  return pl.kernel(
    _pf_body,
    out_type=(jax.ShapeDtypeStruct((2, NP), f32),
              jax.ShapeDtypeStruct((768,), f32)),
    mesh=_mesh(),
    compiler_params=pltpu.CompilerParams(needs_layout_passes=False),
    scratch_types=[
        pltpu.VMEM((ECH, 128), i32), pltpu.VMEM((ECH, 128), i32),
        pltpu.VMEM((NP,), i32), pltpu.VMEM((NT // 128, 128), i32),
        pltpu.VMEM((NT,), f32), pltpu.VMEM((NT,), f32), pltpu.VMEM((NT,), f32),
        pltpu.VMEM((ECH, 128), i32), pltpu.VMEM((ET,), f32),
        pltpu.VMEM((NSL,), f32),
        pltpu.VMEM_SHARED((NP,), f32), pltpu.VMEM_SHARED((128,), f32),
        pltpu.VMEM_SHARED((128,), f32), pltpu.VMEM_SHARED((128,), f32),
    ],
)


def _prep_math(x, W, asv, adv, cb):
    h = lax.dot(x, W, preferred_element_type=f32)
    a_s = jnp.sum(h * asv, axis=1, keepdims=True)
    a_d = jnp.sum(h * adv, axis=1, keepdims=True)
    nr = jnp.sqrt(jnp.sum(cb * cb, axis=1, keepdims=True))
    z = cb / jnp.maximum(nr, 1e-12)
    return h, a_s, a_d, z


def _prep_body(x_ref, W_ref, as_ref, ad_ref, co_ref,
               h_ref, aso_ref, ado_ref, z_ref):
    h, a_s, a_d, z = _prep_math(x_ref[...], W_ref[...], as_ref[...],
                                ad_ref[...], co_ref[...])
    h_ref[...] = h
    aso_ref[...] = a_s
    ado_ref[...] = a_d
    z_ref[...] = z


def _prep(xp, coordp, W, a_src, a_dst):
    return pl.pallas_call(
        _prep_body,
        grid=(GRID,),
        in_specs=[
            pl.BlockSpec((BR, D), lambda i: (i, 0)),
            pl.BlockSpec((D, D), lambda i: (0, 0)),
            pl.BlockSpec((1, D), lambda i: (0, 0)),
            pl.BlockSpec((1, D), lambda i: (0, 0)),
            pl.BlockSpec((BR, 2), lambda i: (i, 0)),
        ],
        out_specs=[
            pl.BlockSpec((BR, D), lambda i: (i, 0)),
            pl.BlockSpec((BR, 1), lambda i: (i, 0)),
            pl.BlockSpec((BR, 1), lambda i: (i, 0)),
            pl.BlockSpec((BR, 2), lambda i: (i, 0)),
        ],
        out_shape=[
            jax.ShapeDtypeStruct((NP, D), f32),
            jax.ShapeDtypeStruct((NP, 1), f32),
            jax.ShapeDtypeStruct((NP, 1), f32),
            jax.ShapeDtypeStruct((NP, 2), f32),
        ],
    )(xp, W, a_src, a_dst, coordp)


def _x1_math(o, s0, s1, a_s, a_d, h0, b, hist, rowid):
    a = a_s + a_d
    a = jnp.where(a > 0, a, 0.2 * a)
    exs = jnp.exp(a)
    rv = jnp.where(rowid < N, 1.0, 0.0)
    x1 = jnp.maximum((o + exs * h0) / (s0 + s1 + exs + 1e-16) + b, 0.0)
    x1 = x1 * rv
    dis = lax.rsqrt(hist + 1.0)
    return x1, dis


def _x1_body(oL0_r, oL1_r, oR0_r, oR1_r, s0_ref, s1_ref, as_ref, ad_ref,
             h0_ref, b_ref, h0c_ref, h1c_ref, x1_ref, dis_ref):
    pid = pl.program_id(0)
    rowid = pid * BR + _fiota((BR, 1), 0)
    o = jnp.concatenate([oL0_r[...] + oL1_r[...], oR0_r[...] + oR1_r[...]],
                        axis=1)
    x1, dis = _x1_math(o, s0_ref[...], s1_ref[...],
                       as_ref[...], ad_ref[...], h0_ref[...], b_ref[...],
                       h0c_ref[...] + h1c_ref[...], rowid)
    x1_ref[...] = x1
    dis_ref[...] = dis


def _x1(oL0, oL1, oR0, oR1, s0, s1, a_s, a_d, h0, b, h0c, h1c):
    col = pl.BlockSpec((BR, 1), lambda i: (i, 0))
    mat = pl.BlockSpec((BR, D), lambda i: (i, 0))
    m64 = pl.BlockSpec((BR, 64), lambda i: (i, 0))
    return pl.pallas_call(
        _x1_body,
        grid=(GRID,),
        in_specs=[m64, m64, m64, m64, col, col, col, col, mat,
                  pl.BlockSpec((1, D), lambda i: (0, 0)), col, col],
        out_specs=[mat, col],
        out_shape=[jax.ShapeDtypeStruct((NP, D), f32),
                   jax.ShapeDtypeStruct((NP, 1), f32)],
    )(oL0, oL1, oR0, oR1, s0, s1, a_s, a_d, h0, b, h0c, h1c)


def _z2_math(a00, a01, a10, a11, z, dis, W1, b1, g1, be1):
    agg0 = a00 + a01
    agg1 = a10 + a11
    aggm = jnp.concatenate([agg0, agg1], axis=1)
    total = aggm + dis * dis * z
    g = lax.dot(total, W1, preferred_element_type=f32) + b1
    return jnp.maximum(g * BNI * g1 + be1, 0.0)


def _z2_body(a00_r, a01_r, a10_r, a11_r, z_r, dis_r, W1_r, b1_r, g1_r, be1_r,
             z2_r):
    z2_r[...] = _z2_math(a00_r[...], a01_r[...], a10_r[...], a11_r[...],
                         z_r[...], dis_r[...], W1_r[...], b1_r[...],
                         g1_r[...], be1_r[...])


def _z2(a00, a01, a10, a11, z, dis, W1p, b1p, g1p, be1p):
    col = pl.BlockSpec((BR, 1), lambda i: (i, 0))
    vec = pl.BlockSpec((1, 64), lambda i: (0, 0))
    return pl.pallas_call(
        _z2_body,
        grid=(GRID,),
        in_specs=[col, col, col, col,
                  pl.BlockSpec((BR, 2), lambda i: (i, 0)), col,
                  pl.BlockSpec((2, 64), lambda i: (0, 0)), vec, vec, vec],
        out_specs=[pl.BlockSpec((BR, 64), lambda i: (i, 0))],
        out_shape=[jax.ShapeDtypeStruct((NP, 64), f32)],
    )(a00, a01, a10, a11, z, dis, W1p, b1p, g1p, be1p)[0]


def _pr_math(a0, a1, z2, dis, W2, b2, g2, be2, Wo, bo, rowid):
    g = lax.dot(a0 + a1 + dis * dis * z2, W2, preferred_element_type=f32) + b2
    z2b = jnp.maximum(g * BNI * g2 + be2, 0.0)
    logits = lax.dot(z2b, Wo, preferred_element_type=f32) + bo
    civ = _fiota((1, D), 1)
    logits = jnp.where(civ < K0, logits, -1e30)
    rmax = jnp.max(logits, axis=1, keepdims=True)
    p = jnp.exp(logits - rmax)
    probs = p / jnp.sum(p, axis=1, keepdims=True)
    rv = rowid < N
    probs = probs * jnp.where(rv, 1.0, 0.0)
    cif = _fiota(logits.shape, 1)
    am = jnp.min(jnp.where(logits == rmax, cif, 1e9), axis=1, keepdims=True)
    cidx = jnp.where(rv, am.astype(i32), NP - NT)
    return probs, cidx


def _pr_body(a0_r, a1_r, z2_r, dis_r, W2_r, b2_r, g2_r, be2_r, Wo_r, bo_r,
             pr_ref, ci_ref):
    pid = pl.program_id(0)
    rowid = pid * BR + _fiota((BR, 1), 0)
    probs, cidx = _pr_math(a0_r[...], a1_r[...], z2_r[...], dis_r[...],
                           W2_r[...], b2_r[...], g2_r[...], be2_r[...],
                           Wo_r[...], bo_r[...], rowid)
    pr_ref[...] = probs
    ci_ref[...] = cidx


def _pr(a0, a1, z2, dis, W2p, b2p, g2p, be2p, Wop, bop):
    col = pl.BlockSpec((BR, 1), lambda i: (i, 0))
    m64 = pl.BlockSpec((BR, 64), lambda i: (i, 0))
    vec = pl.BlockSpec((1, 64), lambda i: (0, 0))
    return pl.pallas_call(
        _pr_body,
        grid=(GRID,),
        in_specs=[m64, m64, m64, col,
                  pl.BlockSpec((64, 64), lambda i: (0, 0)), vec, vec, vec,
                  pl.BlockSpec((64, D), lambda i: (0, 0)),
                  pl.BlockSpec((1, D), lambda i: (0, 0))],
        out_specs=[pl.BlockSpec((BR, D), lambda i: (i, 0)), col],
        out_shape=[jax.ShapeDtypeStruct((NP, D), f32),
                   jax.ShapeDtypeStruct((NP, 1), i32)],
    )(a0, a1, z2, dis, W2p, b2p, g2p, be2p, Wop, bop)


def _nx_body(pr_ref, x1_ref, nx_ref):
    @pl.when(pl.program_id(0) == 0)
    def _():
        nx_ref[...] = jnp.zeros((D, D), f32)
    nx_ref[...] += lax.dot_general(pr_ref[...], x1_ref[...],
                                   (((0,), (0,)), ((), ())),
                                   preferred_element_type=f32)


def _nx(probs, x1):
    return pl.pallas_call(
        _nx_body,
        grid=(GRID,),
        in_specs=[pl.BlockSpec((BR, D), lambda i: (i, 0)),
                  pl.BlockSpec((BR, D), lambda i: (i, 0))],
        out_specs=[pl.BlockSpec((D, D), lambda i: (0, 0))],
        out_shape=[jax.ShapeDtypeStruct((D, D), f32)],
    )(probs, x1)[0]


def _gat_dense(x, mask, W, asv, adv, b, M):
    ri = _fiota((D, D), 0)
    ci = _fiota((D, D), 1)
    h = lax.dot(x, W, preferred_element_type=f32)
    a_col = lax.dot_general(h, asv, (((1,), (1,)), ((), ())),
                            preferred_element_type=f32)
    a_row = lax.dot_general(adv, h, (((1,), (1,)), ((), ())),
                            preferred_element_type=f32)
    e = a_col + a_row
    e = jnp.where(e > 0, e, 0.2 * e)
    eye = jnp.where((ri == ci) & (ci < M), 1.0, 0.0)
    cand = mask + eye
    em = jnp.where(cand > 0, e, -1e30)
    amax = jnp.max(em, axis=0, keepdims=True)
    Wadj = jnp.exp(em - amax)
    ones_col = jnp.ones((D, 1), f32)
    S_col = lax.dot_general(Wadj, ones_col, (((0,), (0,)), ((), ())),
                            preferred_element_type=f32)
    num = lax.dot_general(Wadj, h, (((0,), (0,)), ((), ())),
                          preferred_element_type=f32)
    return num / (S_col + 1e-16) + b


def _dense_math(x, maskraw, curA, curB, W, asv, adv, b,
                W1, b1, g1, be1, W2, b2, g2, be2, Wo, bo, M, K, pool):
    ri = _fiota((D, D), 0)
    ci = _fiota((D, D), 1)
    riv = _fiota((D, 1), 0)
    civ = _fiota((1, D), 1)
    mask = jnp.where((maskraw > 0) & (ri != ci) & (ri < M) & (ci < M), 1.0, 0.0)
    if pool:
        cur = jnp.where(curB > 0, curA / jnp.maximum(curB, 1.0), 0.0)
    else:
        cur = curA
    rv = jnp.where(riv < M, 1.0, 0.0)
    xg = jnp.maximum(_gat_dense(x, mask, W, asv, adv, b, M), 0.0) * rv
    nr = jnp.sqrt(jnp.sum(cur * cur, axis=1, keepdims=True))
    z = cur / jnp.maximum(nr, 1e-12)
    row_any = jnp.max(mask, axis=1, keepdims=True)
    col_any = jnp.max(mask, axis=0, keepdims=True)
    mn0 = jnp.min(jnp.where(row_any > 0, riv, 1e9))
    mn1 = jnp.min(jnp.where(col_any > 0, civ, 1e9))
    P0 = jnp.where(ci == ri + mn0, 1.0, 0.0)
    P1t = jnp.where(ri == ci + mn1, 1.0, 0.0)
    G = lax.dot(P0, lax.dot(mask, P1t, preferred_element_type=f32),
                preferred_element_type=f32)
    ones_col = jnp.ones((D, 1), f32)
    degc = lax.dot_general(G, ones_col, (((0,), (0,)), ((), ())),
                           preferred_element_type=f32) + 1.0
    dis = lax.rsqrt(degc)

    def gcn(hh, b_r):
        t1 = lax.dot_general(G, dis * hh, (((0,), (0,)), ((), ())),
                             preferred_element_type=f32)
        return dis * t1 + dis * dis * hh + b_r

    h1 = lax.dot(z, W1, preferred_element_type=f32)
    z2 = jnp.maximum(gcn(h1, b1) * BNI * g1 + be1, 0.0)
    h2 = lax.dot(z2, W2, preferred_element_type=f32)
    z2b = jnp.maximum(gcn(h2, b2) * BNI * g2 + be2, 0.0)
    logits = lax.dot(z2b, Wo, preferred_element_type=f32) + bo
    logits = jnp.where(civ < K, logits, -1e30)
    rmax = jnp.max(logits, axis=1, keepdims=True)
    p = jnp.exp(logits - rmax)
    probs = p / jnp.sum(p, axis=1, keepdims=True) * rv
    am = jnp.min(jnp.where(logits == rmax, ci, 1e9), axis=1, keepdims=True)
    O = jnp.where((ci == am) & (riv < M), 1.0, 0.0)
    t2 = lax.dot(mask, O, preferred_element_type=f32)
    E2 = lax.dot_general(O, t2, (((0,), (0,)), ((), ())),
                         preferred_element_type=f32)
    mo = jnp.where((E2 > 0) & (ri != ci) & (ri < K) & (ci < K), 1.0, 0.0)
    xo = lax.dot_general(probs, xg, (((0,), (0,)), ((), ())),
                         preferred_element_type=f32)
    sums_p = lax.dot_general(O, cur, (((0,), (0,)), ((), ())),
                             preferred_element_type=f32)
    cntn = lax.dot_general(O, ones_col, (((0,), (0,)), ((), ())),
                           preferred_element_type=f32)
    co = jnp.where(cntn > 0, sums_p / jnp.maximum(cntn, 1.0), 0.0)
    return xo, mo, co


def _dense_body(x_r, m_r, cA_r, cB_r, W_r, as_r, ad_r, b_r,
                W1_r, b1_r, g1_r, be1_r, W2_r, b2_r, g2_r, be2_r,
                Wo_r, bo_r, xo_r, mo_r, co_r, *, M, K, pool):
    xo, mo, co = _dense_math(
        x_r[...], m_r[...], cA_r[...], cB_r[...], W_r[...], as_r[...],
        ad_r[...], b_r[...], W1_r[...], b1_r[...], g1_r[...], be1_r[...],
        W2_r[...], b2_r[...], g2_r[...], be2_r[...], Wo_r[...], bo_r[...],
        M, K, pool)
    xo_r[...] = xo
    mo_r[...] = mo
    co_r[...] = co


def _dense_level(x, mask, curA, curB, conv, pred, M, K, pool):
    full = pl.BlockSpec((D, D), lambda: (0, 0))
    vec = pl.BlockSpec((1, D), lambda: (0, 0))
    colb = pl.BlockSpec((D, 1), lambda: (0, 0))
    Wp = conv['W']
    asv = conv['a_src'][None, :]
    adv = conv['a_dst'][None, :]
    bv = conv['b'][None, :]
    h = pred['W1'].shape[1]
    W1p = jnp.pad(pred['W1'], ((0, D - 2), (0, D - h)))
    W2p = jnp.pad(pred['W2'], ((0, D - h), (0, D - h)))
    Wop = jnp.pad(pred['Wo'], ((0, D - h), (0, D - K)))
    b1p = jnp.pad(pred['b1'], (0, D - h))[None, :]
    g1p = jnp.pad(pred['g1'], (0, D - h))[None, :]
    be1p = jnp.pad(pred['be1'], (0, D - h))[None, :]
    b2p = jnp.pad(pred['b2'], (0, D - h))[None, :]
    g2p = jnp.pad(pred['g2'], (0, D - h))[None, :]
    be2p = jnp.pad(pred['be2'], (0, D - h))[None, :]
    bop = jnp.pad(pred['bo'], (0, D - K))[None, :]
    return pl.pallas_call(
        functools.partial(_dense_body, M=M, K=K, pool=pool),
        in_specs=[full, full, full, colb if pool else full,
                  full, vec, vec, vec,
                  full, vec, vec, vec, full, vec, vec, vec, full, vec],
        out_specs=[full, full, full],
        out_shape=[jax.ShapeDtypeStruct((D, D), f32),
                   jax.ShapeDtypeStruct((D, D), f32),
                   jax.ShapeDtypeStruct((D, D), f32)],
    )(x, mask, curA, curB, Wp, asv, adv, bv, W1p, b1p, g1p, be1p,
      W2p, b2p, g2p, be2p, Wop, bop)


def _final_math(x, maskraw, W, asv, adv, b, M):
    ri = _fiota((D, D), 0)
    ci = _fiota((D, D), 1)
    civ = _fiota((1, D), 1)
    mask = jnp.where((maskraw > 0) & (ri != ci) & (ri < M) & (ci < M), 1.0, 0.0)
    gat = _gat_dense(x, mask, W, asv, adv, b, M)
    rvr = jnp.where(civ < M, 1.0, 0.0)
    return lax.dot_general(rvr, gat, (((1,), (0,)), ((), ())),
                           preferred_element_type=f32) / M


def _final_body(x_r, m_r, W_r, as_r, ad_r, b_r, o_r, *, M):
    o_r[...] = _final_math(x_r[...], m_r[...], W_r[...], as_r[...],
                           ad_r[...], b_r[...], M)


def _final_level(x, mask, conv, M):
    full = pl.BlockSpec((D, D), lambda: (0, 0))
    vec = pl.BlockSpec((1, D), lambda: (0, 0))
    return pl.pallas_call(
        functools.partial(_final_body, M=M),
        in_specs=[full, full, full, vec, vec, vec],
        out_specs=[pl.BlockSpec((1, D), lambda: (0, 0))],
        out_shape=[jax.ShapeDtypeStruct((1, D), f32)],
    )(x, mask, conv['W'], conv['a_src'][None, :], conv['a_dst'][None, :],
      conv['b'][None, :])[0]


def kernel(x, edge_index, batch, coord, params):
    xp = jnp.pad(x, ((0, NP - N), (0, 0)))
    coordp = jnp.pad(coord, ((0, NP - N), (0, 0)))
    src = edge_index[0].astype(i32)
    dst = edge_index[1].astype(i32)
    mn = jnp.min(edge_index, axis=1).astype(i32)
    dump = (N + 2000 + (jnp.arange(EP - E, dtype=i32) % (NP - N - 2000)))

    def pad_e(a):
        return jnp.concatenate([a, dump]).reshape(ERW, 128)

    src2 = pad_e(src)
    dst2 = pad_e(dst)
    srcs2 = pad_e(src - mn[0])
    dsts2 = pad_e(dst - mn[1])

    p0 = params['conv0']
    h0, a_s, a_d, z = _prep(xp, coordp, p0['W'], p0['a_src'][None, :],
                            p0['a_dst'][None, :])

    ex, s2, hist2 = _pass_a()(src2, dst2, dsts2, a_s.reshape(NP),
                            a_d.reshape(NP))
    oL = _pass_rows()(src2, dst2, ex, h0[:, :64])
    oR = _pass_rows()(src2, dst2, ex, h0[:, 64:])
    x1, dis = _x1(oL[0], oL[1], oR[0], oR[1],
                  s2[0].reshape(NP, 1), s2[1].reshape(NP, 1),
                  a_s, a_d, h0, p0['b'][None, :],
                  hist2[0].reshape(NP, 1), hist2[1].reshape(NP, 1))

    nrm, agg = _pass_d()(srcs2, dsts2, dis.reshape(NP), z[:, 0], z[:, 1])

    pr0 = params['pred0']
    h = pr0['W1'].shape[1]
    W1p = jnp.pad(pr0['W1'], ((0, 0), (0, 64 - h)))
    z2 = _z2(agg[0, 0].reshape(NP, 1), agg[1, 0].reshape(NP, 1),
             agg[0, 1].reshape(NP, 1), agg[1, 1].reshape(NP, 1),
             z, dis, W1p, jnp.pad(pr0['b1'], (0, 64 - h))[None, :],
             jnp.pad(pr0['g1'], (0, 64 - h))[None, :],
             jnp.pad(pr0['be1'], (0, 64 - h))[None, :])

    agg2 = _pass_rows()(srcs2, dsts2, nrm, z2)

    W2p = jnp.pad(pr0['W2'], ((0, 64 - h), (0, 64 - h)))
    Wop = jnp.pad(pr0['Wo'], ((0, 64 - h), (0, D - K0)))
    probs, cidx = _pr(agg2[0], agg2[1], z2, dis, W2p,
                      jnp.pad(pr0['b2'], (0, 64 - h))[None, :],
                      jnp.pad(pr0['g2'], (0, 64 - h))[None, :],
                      jnp.pad(pr0['be2'], (0, 64 - h))[None, :],
                      Wop, jnp.pad(pr0['bo'], (0, D - K0))[None, :])

    cidx_f = cidx.reshape(NP)
    cnt, pool = _pass_f()(src2, dst2, cidx_f, coordp[:, 0], coordp[:, 1])
    nx = _nx(probs, x1)

    cnt_t = cnt[0] + cnt[1]
    mask1 = jnp.pad(cnt_t[:K0 * K0].reshape(K0, K0),
                    ((0, D - K0), (0, D - K0)))
    pool_r = pool.reshape(2, 3, 128)
    pool_t = pool_r[0] + pool_r[1]
    sums = jnp.pad(pool_t[0:2].T, ((0, 0), (0, D - 2)))
    cnt_p = pool_t[2].reshape(D, 1)

    x2, mask2, cur2 = _dense_level(nx, mask1, sums, cnt_p,
                                   params['conv1'], params['pred1'],
                                   M=100, K=50, pool=True)
    x3, mask3, cur3 = _dense_level(x2, mask2, cur2, cur2,
                                   params['conv2'], params['pred2'],
                                   M=50, K=10, pool=False)
    return _final_level(x3, mask3, params['conv3'], M=10)
